# Initial kernel scaffold; baseline (speedup 1.0000x reference)
import functools

import jax
import jax.numpy as jnp
from jax import lax
from jax.experimental import pallas as pl
from jax.experimental.pallas import tpu as pltpu
from jax.experimental.pallas import tpu_sc as plsc

N_NODES = 10000
E_EDGES = 320000
FDIM = 128
EROWS = E_EDGES // 128
NC, NS = 2, 16
NW = NC * NS
CHUNKS = -(-EROWS // NW)
RPT = N_NODES // NS
ZCH = 125

_MESH = plsc.VectorSubcoreMesh(
    core_axis_name="c", subcore_axis_name="s", num_cores=NC, num_subcores=NS)


def _segsum_cnt_body(table, src2d, dst2d, z128, z16, ones_h,
                     acc_out, cnt_out,
                     srow, drow, rows, zbuf, sem, ones_v, cbuf,
                     acc_sh, cnt_sh):
    cid = lax.axis_index("c")
    sid = lax.axis_index("s")
    wid = sid * NC + cid
    r0 = sid * RPT

    pltpu.sync_copy(z128, zbuf)
    for i in range(RPT // ZCH):
        pltpu.sync_copy(zbuf, acc_sh.at[pl.ds(r0 + i * ZCH, ZCH)])
    pltpu.sync_copy(z16, cbuf)
    pltpu.sync_copy(cbuf, cnt_sh.at[pl.ds(r0, RPT)])
    pltpu.sync_copy(ones_h, ones_v)
    plsc.subcore_barrier()

    def chunk(k, carry):
        t = wid + k * NW

        @pl.when(t < EROWS)
        def _():
            pltpu.sync_copy(src2d.at[t], srow)
            pltpu.sync_copy(dst2d.at[t], drow)
            pltpu.async_copy(table.at[srow], rows, sem).wait()
            pltpu.sync_copy(rows, acc_sh.at[drow], add=True)
            pltpu.sync_copy(ones_v, cnt_sh.at[drow], add=True)
        return carry

    lax.fori_loop(0, CHUNKS, chunk, 0)
    plsc.subcore_barrier()

    for i in range(RPT // ZCH):
        sl = pl.ds(r0 + i * ZCH, ZCH)
        pltpu.sync_copy(acc_sh.at[sl], zbuf)
        pltpu.sync_copy(zbuf, acc_out.at[cid, sl])
    pltpu.sync_copy(cnt_sh.at[pl.ds(r0, RPT)], cbuf)
    pltpu.sync_copy(cbuf, cnt_out.at[cid, pl.ds(r0, RPT)])


def _segsum_body(table, src2d, dst2d, z128,
                 acc_out,
                 srow, drow, rows, zbuf, sem,
                 acc_sh):
    cid = lax.axis_index("c")
    sid = lax.axis_index("s")
    wid = sid * NC + cid
    r0 = sid * RPT

    pltpu.sync_copy(z128, zbuf)
    for i in range(RPT // ZCH):
        pltpu.sync_copy(zbuf, acc_sh.at[pl.ds(r0 + i * ZCH, ZCH)])
    plsc.subcore_barrier()

    def chunk(k, carry):
        t = wid + k * NW

        @pl.when(t < EROWS)
        def _():
            pltpu.sync_copy(src2d.at[t], srow)
            pltpu.sync_copy(dst2d.at[t], drow)
            pltpu.async_copy(table.at[srow], rows, sem).wait()
            pltpu.sync_copy(rows, acc_sh.at[drow], add=True)
        return carry

    lax.fori_loop(0, CHUNKS, chunk, 0)
    plsc.subcore_barrier()

    for i in range(RPT // ZCH):
        sl = pl.ds(r0 + i * ZCH, ZCH)
        pltpu.sync_copy(acc_sh.at[sl], zbuf)
        pltpu.sync_copy(zbuf, acc_out.at[cid, sl])


_segsum_cnt = functools.partial(
    pl.kernel, _segsum_cnt_body,
    out_type=(jax.ShapeDtypeStruct((NC, N_NODES, FDIM), jnp.float32),
              jax.ShapeDtypeStruct((NC, N_NODES, 16), jnp.float32)),
    mesh=_MESH,
    scratch_types=(
        pltpu.VMEM((128,), jnp.int32),
        pltpu.VMEM((128,), jnp.int32),
        pltpu.VMEM((128, FDIM), jnp.float32),
        pltpu.VMEM((ZCH, FDIM), jnp.float32),
        pltpu.SemaphoreType.DMA,
        pltpu.VMEM((128, 16), jnp.float32),
        pltpu.VMEM((RPT, 16), jnp.float32),
        pltpu.VMEM_SHARED((N_NODES, FDIM), jnp.float32),
        pltpu.VMEM_SHARED((N_NODES, 16), jnp.float32),
    ),
)()

_segsum = functools.partial(
    pl.kernel, _segsum_body,
    out_type=jax.ShapeDtypeStruct((NC, N_NODES, FDIM), jnp.float32),
    mesh=_MESH,
    scratch_types=(
        pltpu.VMEM((128,), jnp.int32),
        pltpu.VMEM((128,), jnp.int32),
        pltpu.VMEM((128, FDIM), jnp.float32),
        pltpu.VMEM((ZCH, FDIM), jnp.float32),
        pltpu.SemaphoreType.DMA,
        pltpu.VMEM_SHARED((N_NODES, FDIM), jnp.float32),
    ),
)()


def _escore_body(su_h, sv_h, src2d, dst2d,
                 out_h,
                 su_v, sv_v, srow, drow, outv):
    cid = lax.axis_index("c")
    sid = lax.axis_index("s")
    wid = sid * NC + cid

    pltpu.sync_copy(su_h, su_v)
    pltpu.sync_copy(sv_h, sv_v)
    ii = lax.iota(jnp.int32, 16)
    q = ii // 4
    col = ii - q * 4

    def chunk(k, carry):
        t = wid + k * NW

        @pl.when(t < EROWS)
        def _():
            pltpu.sync_copy(src2d.at[t], srow)
            pltpu.sync_copy(dst2d.at[t], drow)
            for j in range(32):
                e0 = jnp.int32(j * 4)
                sids = plsc.load_gather(srow, [e0 + q])
                dids = plsc.load_gather(drow, [e0 + q])
                a = plsc.load_gather(su_v, [sids, col])
                b = plsc.load_gather(sv_v, [dids, col])
                outv[pl.ds(j * 16, 16)] = a + b
            pltpu.sync_copy(outv, out_h.at[t])
        return carry

    lax.fori_loop(0, CHUNKS, chunk, 0)


_escore = functools.partial(
    pl.kernel, _escore_body,
    out_type=jax.ShapeDtypeStruct((EROWS, 512), jnp.float32),
    mesh=_MESH,
    scratch_types=(
        pltpu.VMEM((N_NODES, 4), jnp.float32),
        pltpu.VMEM((N_NODES, 4), jnp.float32),
        pltpu.VMEM((128,), jnp.int32),
        pltpu.VMEM((128,), jnp.int32),
        pltpu.VMEM((512,), jnp.float32),
    ),
)()


BLK = 400


def _tc1_body(x_ref, acc_ref, cnt_ref, ws_ref, wn_ref, b_ref, o_ref):
    agg = acc_ref[0] + acc_ref[1]
    cnt = cnt_ref[0, :, 0:1] + cnt_ref[1, :, 0:1]
    mean = agg / jnp.maximum(cnt, 1.0)
    h = (jnp.dot(x_ref[...], ws_ref[...], preferred_element_type=jnp.float32)
         + jnp.dot(mean, wn_ref[...], preferred_element_type=jnp.float32)
         + b_ref[...])
    o_ref[...] = jnp.maximum(h, 0.0)


def _tc2_body(h_ref, acc_ref, cnt_ref, ws_ref, wn_ref, b_ref, wc_ref, b8_ref,
              o_ref):
    agg = acc_ref[0] + acc_ref[1]
    cnt = cnt_ref[0, :, 0:1] + cnt_ref[1, :, 0:1]
    mean = agg / jnp.maximum(cnt, 1.0)
    h2 = (jnp.dot(h_ref[...], ws_ref[...], preferred_element_type=jnp.float32)
          + jnp.dot(mean, wn_ref[...], preferred_element_type=jnp.float32)
          + b_ref[...])
    o_ref[...] = (jnp.dot(h2, wc_ref[...], preferred_element_type=jnp.float32)
                  + b8_ref[...])


def _tc1(x, acc, cnt, ws, wn, b):
    return pl.pallas_call(
        _tc1_body,
        grid=(N_NODES // BLK,),
        in_specs=[
            pl.BlockSpec((BLK, FDIM), lambda i: (i, 0)),
            pl.BlockSpec((NC, BLK, FDIM), lambda i: (0, i, 0)),
            pl.BlockSpec((NC, BLK, 16), lambda i: (0, i, 0)),
            pl.BlockSpec((FDIM, FDIM), lambda i: (0, 0)),
            pl.BlockSpec((FDIM, FDIM), lambda i: (0, 0)),
            pl.BlockSpec((1, FDIM), lambda i: (0, 0)),
        ],
        out_specs=pl.BlockSpec((BLK, FDIM), lambda i: (i, 0)),
        out_shape=jax.ShapeDtypeStruct((N_NODES, FDIM), jnp.float32),
    )(x, acc, cnt, ws, wn, b)


def _tc2(h, acc, cnt, ws, wn, b, wc, b8):
    return pl.pallas_call(
        _tc2_body,
        grid=(N_NODES // BLK,),
        in_specs=[
            pl.BlockSpec((BLK, FDIM), lambda i: (i, 0)),
            pl.BlockSpec((NC, BLK, FDIM), lambda i: (0, i, 0)),
            pl.BlockSpec((NC, BLK, 16), lambda i: (0, i, 0)),
            pl.BlockSpec((FDIM, FDIM), lambda i: (0, 0)),
            pl.BlockSpec((FDIM, FDIM), lambda i: (0, 0)),
            pl.BlockSpec((1, FDIM), lambda i: (0, 0)),
            pl.BlockSpec((FDIM, 8), lambda i: (0, 0)),
            pl.BlockSpec((1, 8), lambda i: (0, 0)),
        ],
        out_specs=pl.BlockSpec((BLK, 8), lambda i: (i, 0)),
        out_shape=jax.ShapeDtypeStruct((N_NODES, 8), jnp.float32),
    )(h, acc, cnt, ws, wn, b, wc, b8)


def kernel(x, edge_index, e, W1_self, W1_neigh, b1, W2_self, W2_neigh, b2,
           Wp, bp):
    del e
    src2d = edge_index[0].reshape(EROWS, 128)
    dst2d = edge_index[1].reshape(EROWS, 128)
    z128 = jnp.zeros((ZCH, FDIM), jnp.float32)
    z16 = jnp.zeros((RPT, 16), jnp.float32)
    ones16 = jnp.ones((128, 16), jnp.float32)

    acc1, cnt = _segsum_cnt(x, src2d, dst2d, z128, z16, ones16)
    h1 = _tc1(x, acc1, cnt, W1_self, W1_neigh, b1.reshape(1, FDIM))
    acc2 = _segsum(h1, src2d, dst2d, z128)

    wu = jnp.pad(Wp[:FDIM], ((0, 0), (0, 1)))
    wv = jnp.pad(Wp[FDIM:], ((0, 0), (0, 1)))
    wc = jnp.concatenate([wu, wv], axis=1)
    b8 = jnp.concatenate([jnp.zeros((4,), jnp.float32), bp,
                          jnp.zeros((1,), jnp.float32)]).reshape(1, 8)
    suv = _tc2(h1, acc2, cnt, W2_self, W2_neigh, b2.reshape(1, FDIM), wc, b8)

    su4 = suv[:, :4]
    sv4 = suv[:, 4:]
    out4 = _escore(su4, sv4, src2d, dst2d)
    return out4.reshape(E_EDGES, 4)[:, :3]


# speedup vs baseline: 4.6010x; 4.6010x over previous
import functools

import jax
import jax.numpy as jnp
from jax import lax
from jax.experimental import pallas as pl
from jax.experimental.pallas import tpu as pltpu
from jax.experimental.pallas import tpu_sc as plsc

N_NODES = 10000
E_EDGES = 320000
FDIM = 128
EROWS = E_EDGES // 128
NC, NS = 2, 16
NW = NC * NS
CHUNKS = -(-EROWS // NW)
RPT = 640
ZCH = 80

_MESH = plsc.VectorSubcoreMesh(
    core_axis_name="c", subcore_axis_name="s", num_cores=NC, num_subcores=NS)


def _node_slices(sid):
    r0 = sid * RPT
    nrows = jnp.where(sid == NS - 1, 400, RPT)
    return r0, nrows


def _segsum_body(table, src1, dst1, z128,
                 acc_out,
                 srow, drow, rows, zbuf, sem,
                 acc_sh):
    cid = lax.axis_index("c")
    sid = lax.axis_index("s")
    wid = sid * NC + cid
    r0, nrows = _node_slices(sid)

    pltpu.sync_copy(z128, zbuf)
    for i in range(RPT // ZCH):
        @pl.when(i * ZCH < nrows)
        def _():
            off = pl.multiple_of(r0 + i * ZCH, 8)
            pltpu.sync_copy(zbuf, acc_sh.at[pl.ds(off, ZCH)])
    plsc.subcore_barrier()

    def chunk(k, carry):
        t = wid + k * NW

        @pl.when(t < EROWS)
        def _():
            eoff = pl.multiple_of(t * 128, 128)
            pltpu.sync_copy(src1.at[pl.ds(eoff, 128)], srow)
            pltpu.sync_copy(dst1.at[pl.ds(eoff, 128)], drow)
            pltpu.async_copy(table.at[srow], rows, sem).wait()
            pltpu.sync_copy(rows, acc_sh.at[drow], add=True)
        return carry

    lax.fori_loop(0, CHUNKS, chunk, 0)
    plsc.subcore_barrier()

    for i in range(RPT // ZCH):
        @pl.when(i * ZCH < nrows)
        def _():
            off = pl.multiple_of(r0 + i * ZCH, 8)
            hoff = pl.multiple_of(cid * N_NODES + r0 + i * ZCH, 8)
            pltpu.sync_copy(acc_sh.at[pl.ds(off, ZCH)], zbuf)
            pltpu.sync_copy(zbuf, acc_out.at[pl.ds(hoff, ZCH)])


_segsum = functools.partial(
    pl.kernel, _segsum_body,
    out_type=jax.ShapeDtypeStruct((NC * N_NODES, FDIM), jnp.float32),
    mesh=_MESH,
    compiler_params=pltpu.CompilerParams(needs_layout_passes=False),
    scratch_types=(
        pltpu.VMEM((128,), jnp.int32),
        pltpu.VMEM((128,), jnp.int32),
        pltpu.VMEM((128, FDIM), jnp.float32),
        pltpu.VMEM((ZCH, FDIM), jnp.float32),
        pltpu.SemaphoreType.DMA,
        pltpu.VMEM_SHARED((N_NODES, FDIM), jnp.float32),
    ),
)()


def _cnt_body(dst1, z128, ones_h,
              cnt_out,
              drow, zbuf, ones_v,
              cnt_sh):
    cid = lax.axis_index("c")
    sid = lax.axis_index("s")
    wid = sid * NC + cid
    r0, nrows = _node_slices(sid)

    pltpu.sync_copy(z128, zbuf)
    pltpu.sync_copy(ones_h, ones_v)
    for i in range(RPT // ZCH):
        @pl.when(i * ZCH < nrows)
        def _():
            off = pl.multiple_of(r0 + i * ZCH, 8)
            pltpu.sync_copy(zbuf, cnt_sh.at[pl.ds(off, ZCH)])
    plsc.subcore_barrier()

    def chunk(k, carry):
        t = wid + k * NW

        @pl.when(t < EROWS)
        def _():
            eoff = pl.multiple_of(t * 128, 128)
            pltpu.sync_copy(dst1.at[pl.ds(eoff, 128)], drow)
            pltpu.sync_copy(ones_v, cnt_sh.at[drow], add=True)
        return carry

    lax.fori_loop(0, CHUNKS, chunk, 0)
    plsc.subcore_barrier()

    for i in range(RPT // ZCH):
        @pl.when(i * ZCH < nrows)
        def _():
            off = pl.multiple_of(r0 + i * ZCH, 8)
            hoff = pl.multiple_of(cid * N_NODES + r0 + i * ZCH, 8)
            pltpu.sync_copy(cnt_sh.at[pl.ds(off, ZCH)], zbuf)
            pltpu.sync_copy(zbuf, cnt_out.at[pl.ds(hoff, ZCH)])


_cnt_kernel = functools.partial(
    pl.kernel, _cnt_body,
    out_type=jax.ShapeDtypeStruct((NC * N_NODES, FDIM), jnp.float32),
    mesh=_MESH,
    compiler_params=pltpu.CompilerParams(needs_layout_passes=False),
    scratch_types=(
        pltpu.VMEM((128,), jnp.int32),
        pltpu.VMEM((ZCH, FDIM), jnp.float32),
        pltpu.VMEM((128, FDIM), jnp.float32),
        pltpu.VMEM_SHARED((N_NODES, FDIM), jnp.float32),
    ),
)()


def _escore_body(suv_h, src1, dst1, idxtab_h,
                 out_h,
                 suv_v, srow, drow, outv, idx_v):
    cid = lax.axis_index("c")
    sid = lax.axis_index("s")
    wid = sid * NC + cid

    pltpu.sync_copy(suv_h, suv_v)
    pltpu.sync_copy(idxtab_h, idx_v)

    def chunk(k, carry):
        t = wid + k * NW

        @pl.when(t < EROWS)
        def _():
            eoff = pl.multiple_of(t * 128, 128)
            pltpu.sync_copy(src1.at[pl.ds(eoff, 128)], srow)
            pltpu.sync_copy(dst1.at[pl.ds(eoff, 128)], drow)
            col = idx_v[pl.ds(32 * 16, 16)]
            col4 = idx_v[pl.ds(33 * 16, 16)]
            eight = idx_v[pl.ds(34 * 16, 16)]
            for j in range(32):
                eidx = idx_v[pl.ds(j * 16, 16)]
                sids = plsc.load_gather(srow, [eidx])
                dids = plsc.load_gather(drow, [eidx])
                a = plsc.load_gather(suv_v, [sids * eight + col])
                b = plsc.load_gather(suv_v, [dids * eight + col4])
                outv[pl.ds(j * 16, 16)] = a + b
            ooff = pl.multiple_of(t * 512, 512)
            pltpu.sync_copy(outv, out_h.at[pl.ds(ooff, 512)])
        return carry

    lax.fori_loop(0, CHUNKS, chunk, 0)


_escore = functools.partial(
    pl.kernel, _escore_body,
    out_type=jax.ShapeDtypeStruct((E_EDGES * 4,), jnp.float32),
    mesh=_MESH,
    compiler_params=pltpu.CompilerParams(needs_layout_passes=False),
    scratch_types=(
        pltpu.VMEM((N_NODES * 8,), jnp.float32),
        pltpu.VMEM((128,), jnp.int32),
        pltpu.VMEM((128,), jnp.int32),
        pltpu.VMEM((512,), jnp.float32),
        pltpu.VMEM((35 * 16,), jnp.int32),
    ),
)()


BLK = 400


def _tc1_body(x_ref, acc_ref, cnt_ref, ws_ref, wn_ref, b_ref, o_ref):
    agg = acc_ref[0] + acc_ref[1]
    cnt = cnt_ref[0, :, 0:1] + cnt_ref[1, :, 0:1]
    mean = agg / jnp.maximum(cnt, 1.0)
    h = (jnp.dot(x_ref[...], ws_ref[...], preferred_element_type=jnp.float32)
         + jnp.dot(mean, wn_ref[...], preferred_element_type=jnp.float32)
         + b_ref[...])
    o_ref[...] = jnp.maximum(h, 0.0)


def _tc2_body(h_ref, acc_ref, cnt_ref, ws_ref, wn_ref, b_ref, wc_ref, b8_ref,
              o_ref):
    agg = acc_ref[0] + acc_ref[1]
    cnt = cnt_ref[0, :, 0:1] + cnt_ref[1, :, 0:1]
    mean = agg / jnp.maximum(cnt, 1.0)
    h2 = (jnp.dot(h_ref[...], ws_ref[...], preferred_element_type=jnp.float32)
          + jnp.dot(mean, wn_ref[...], preferred_element_type=jnp.float32)
          + b_ref[...])
    o_ref[...] = (jnp.dot(h2, wc_ref[...], preferred_element_type=jnp.float32)
                  + b8_ref[...])


def _tc1(x, acc, cnt, ws, wn, b):
    return pl.pallas_call(
        _tc1_body,
        grid=(N_NODES // BLK,),
        in_specs=[
            pl.BlockSpec((BLK, FDIM), lambda i: (i, 0)),
            pl.BlockSpec((NC, BLK, FDIM), lambda i: (0, i, 0)),
            pl.BlockSpec((NC, BLK, FDIM), lambda i: (0, i, 0)),
            pl.BlockSpec((FDIM, FDIM), lambda i: (0, 0)),
            pl.BlockSpec((FDIM, FDIM), lambda i: (0, 0)),
            pl.BlockSpec((1, FDIM), lambda i: (0, 0)),
        ],
        out_specs=pl.BlockSpec((BLK, FDIM), lambda i: (i, 0)),
        out_shape=jax.ShapeDtypeStruct((N_NODES, FDIM), jnp.float32),
    )(x, acc, cnt, ws, wn, b)


def _tc2(h, acc, cnt, ws, wn, b, wc, b8):
    return pl.pallas_call(
        _tc2_body,
        grid=(N_NODES // BLK,),
        in_specs=[
            pl.BlockSpec((BLK, FDIM), lambda i: (i, 0)),
            pl.BlockSpec((NC, BLK, FDIM), lambda i: (0, i, 0)),
            pl.BlockSpec((NC, BLK, FDIM), lambda i: (0, i, 0)),
            pl.BlockSpec((FDIM, FDIM), lambda i: (0, 0)),
            pl.BlockSpec((FDIM, FDIM), lambda i: (0, 0)),
            pl.BlockSpec((1, FDIM), lambda i: (0, 0)),
            pl.BlockSpec((FDIM, 8), lambda i: (0, 0)),
            pl.BlockSpec((1, 8), lambda i: (0, 0)),
        ],
        out_specs=pl.BlockSpec((BLK, 8), lambda i: (i, 0)),
        out_shape=jax.ShapeDtypeStruct((N_NODES, 8), jnp.float32),
    )(h, acc, cnt, ws, wn, b, wc, b8)


def kernel(x, edge_index, e, W1_self, W1_neigh, b1, W2_self, W2_neigh, b2,
           Wp, bp):
    del e
    src1 = edge_index[0]
    dst1 = edge_index[1]
    z128 = jnp.zeros((ZCH, FDIM), jnp.float32)
    ones128 = jnp.ones((128, FDIM), jnp.float32)
    idxtab = jnp.asarray(
        [j * 4 + l // 4 for j in range(32) for l in range(16)]
        + [l % 4 for l in range(16)]
        + [4 + l % 4 for l in range(16)]
        + [8] * 16, jnp.int32)

    cnt = _cnt_kernel(dst1, z128, ones128).reshape(NC, N_NODES, FDIM)
    acc1 = _segsum(x, src1, dst1, z128).reshape(NC, N_NODES, FDIM)
    h1 = _tc1(x, acc1, cnt, W1_self, W1_neigh, b1.reshape(1, FDIM))
    acc2 = _segsum(h1, src1, dst1, z128).reshape(NC, N_NODES, FDIM)

    wu = jnp.pad(Wp[:FDIM], ((0, 0), (0, 1)))
    wv = jnp.pad(Wp[FDIM:], ((0, 0), (0, 1)))
    wc = jnp.concatenate([wu, wv], axis=1)
    b8 = jnp.concatenate([jnp.zeros((4,), jnp.float32), bp,
                          jnp.zeros((1,), jnp.float32)]).reshape(1, 8)
    suv = _tc2(h1, acc2, cnt, W2_self, W2_neigh, b2.reshape(1, FDIM), wc, b8)

    out4 = _escore(suv.reshape(-1), src1, dst1, idxtab)
    return out4.reshape(E_EDGES, 4)[:, :3]


# speedup vs baseline: 5.6304x; 1.2237x over previous
import functools

import jax
import jax.numpy as jnp
from jax import lax
from jax.experimental import pallas as pl
from jax.experimental.pallas import tpu as pltpu
from jax.experimental.pallas import tpu_sc as plsc

N_NODES = 10000
E_EDGES = 320000
FDIM = 128
EROWS = E_EDGES // 128
NC, NS = 2, 16
NW = NC * NS
CHUNKS = -(-EROWS // NW)
RPT = 640
ZCH = 80

_MESH = plsc.VectorSubcoreMesh(
    core_axis_name="c", subcore_axis_name="s", num_cores=NC, num_subcores=NS)


def _node_slices(sid):
    r0 = sid * RPT
    nrows = jnp.where(sid == NS - 1, 400, RPT)
    return r0, nrows


NBUF = 2


def _segsum_body(table, src1, dst1, z128,
                 acc_out,
                 srow, drow, rows, zbuf, semi, semg, sems,
                 acc_sh):
    cid = lax.axis_index("c")
    sid = lax.axis_index("s")
    wid = sid * NC + cid
    r0, nrows = _node_slices(sid)

    pltpu.sync_copy(z128, zbuf)
    for i in range(RPT // ZCH):
        @pl.when(i * ZCH < nrows)
        def _():
            off = pl.multiple_of(r0 + i * ZCH, 8)
            pltpu.sync_copy(zbuf, acc_sh.at[pl.ds(off, ZCH)])
    plsc.subcore_barrier()

    def chunk4(kk, carry):
        base = wid + kk * (NBUF * NW)
        for q in range(NBUF):
            t = base + q * NW

            @pl.when(t < EROWS)
            def _(t=t, q=q):
                eoff = pl.multiple_of(t * 128, 128)
                pltpu.async_copy(src1.at[pl.ds(eoff, 128)], srow.at[q], semi)
                pltpu.async_copy(dst1.at[pl.ds(eoff, 128)], drow.at[q], semi)
        for q in range(NBUF):
            t = base + q * NW

            @pl.when(t < EROWS)
            def _(t=t, q=q):
                eoff = pl.multiple_of(t * 128, 128)
                pltpu.make_async_copy(
                    src1.at[pl.ds(eoff, 128)], srow.at[q], semi).wait()
                pltpu.make_async_copy(
                    dst1.at[pl.ds(eoff, 128)], drow.at[q], semi).wait()
                pltpu.async_copy(table.at[srow.at[q]],
                                 rows.at[pl.ds(q * 128, 128)], semg)
        for q in range(NBUF):
            t = base + q * NW

            @pl.when(t < EROWS)
            def _(t=t, q=q):
                pltpu.make_async_copy(
                    table.at[srow.at[q]],
                    rows.at[pl.ds(q * 128, 128)], semg).wait()
                pltpu.async_copy(rows.at[pl.ds(q * 128, 128)],
                                 acc_sh.at[drow.at[q]], sems, add=True)
        for q in range(NBUF):
            t = base + q * NW

            @pl.when(t < EROWS)
            def _(t=t, q=q):
                pltpu.make_async_copy(
                    rows.at[pl.ds(q * 128, 128)],
                    acc_sh.at[drow.at[q]], sems).wait()
        return carry

    lax.fori_loop(0, -(-CHUNKS // NBUF), chunk4, 0)
    plsc.subcore_barrier()

    for i in range(RPT // ZCH):
        @pl.when(i * ZCH < nrows)
        def _():
            off = pl.multiple_of(r0 + i * ZCH, 8)
            hoff = pl.multiple_of(cid * N_NODES + r0 + i * ZCH, 8)
            pltpu.sync_copy(acc_sh.at[pl.ds(off, ZCH)], zbuf)
            pltpu.sync_copy(zbuf, acc_out.at[pl.ds(hoff, ZCH)])


_segsum = functools.partial(
    pl.kernel, _segsum_body,
    out_type=jax.ShapeDtypeStruct((NC * N_NODES, FDIM), jnp.float32),
    mesh=_MESH,
    compiler_params=pltpu.CompilerParams(needs_layout_passes=False),
    scratch_types=(
        pltpu.VMEM((NBUF, 128), jnp.int32),
        pltpu.VMEM((NBUF, 128), jnp.int32),
        pltpu.VMEM((NBUF * 128, FDIM), jnp.float32),
        pltpu.VMEM((ZCH, FDIM), jnp.float32),
        pltpu.SemaphoreType.DMA,
        pltpu.SemaphoreType.DMA,
        pltpu.SemaphoreType.DMA,
        pltpu.VMEM_SHARED((N_NODES, FDIM), jnp.float32),
    ),
)()


def _cnt_body(dst1, z128, ones_h,
              cnt_out,
              drow, zbuf, ones_v,
              cnt_sh):
    cid = lax.axis_index("c")
    sid = lax.axis_index("s")
    wid = sid * NC + cid
    r0, nrows = _node_slices(sid)

    pltpu.sync_copy(z128, zbuf)
    pltpu.sync_copy(ones_h, ones_v)
    for i in range(RPT // ZCH):
        @pl.when(i * ZCH < nrows)
        def _():
            off = pl.multiple_of(r0 + i * ZCH, 8)
            pltpu.sync_copy(zbuf, cnt_sh.at[pl.ds(off, ZCH)])
    plsc.subcore_barrier()

    def chunk(k, carry):
        t = wid + k * NW

        @pl.when(t < EROWS)
        def _():
            eoff = pl.multiple_of(t * 128, 128)
            pltpu.sync_copy(dst1.at[pl.ds(eoff, 128)], drow)
            pltpu.sync_copy(ones_v, cnt_sh.at[drow], add=True)
        return carry

    lax.fori_loop(0, CHUNKS, chunk, 0)
    plsc.subcore_barrier()

    for i in range(RPT // ZCH):
        @pl.when(i * ZCH < nrows)
        def _():
            off = pl.multiple_of(r0 + i * ZCH, 8)
            hoff = pl.multiple_of(cid * N_NODES + r0 + i * ZCH, 8)
            pltpu.sync_copy(cnt_sh.at[pl.ds(off, ZCH)], zbuf)
            pltpu.sync_copy(zbuf, cnt_out.at[pl.ds(hoff, ZCH)])


_cnt_kernel = functools.partial(
    pl.kernel, _cnt_body,
    out_type=jax.ShapeDtypeStruct((NC * N_NODES, FDIM), jnp.float32),
    mesh=_MESH,
    compiler_params=pltpu.CompilerParams(needs_layout_passes=False),
    scratch_types=(
        pltpu.VMEM((128,), jnp.int32),
        pltpu.VMEM((ZCH, FDIM), jnp.float32),
        pltpu.VMEM((128, FDIM), jnp.float32),
        pltpu.VMEM_SHARED((N_NODES, FDIM), jnp.float32),
    ),
)()


def _escore_body(suv_h, src1, dst1, idxtab_h,
                 out_h,
                 suv_v, srow, drow, outv, idx_v):
    cid = lax.axis_index("c")
    sid = lax.axis_index("s")
    wid = sid * NC + cid

    pltpu.sync_copy(suv_h, suv_v)
    pltpu.sync_copy(idxtab_h, idx_v)

    def chunk(k, carry):
        t = wid + k * NW

        @pl.when(t < EROWS)
        def _():
            eoff = pl.multiple_of(t * 128, 128)
            pltpu.sync_copy(src1.at[pl.ds(eoff, 128)], srow)
            pltpu.sync_copy(dst1.at[pl.ds(eoff, 128)], drow)
            col = idx_v[pl.ds(32 * 16, 16)]
            col4 = idx_v[pl.ds(33 * 16, 16)]
            eight = idx_v[pl.ds(34 * 16, 16)]
            for j in range(32):
                eidx = idx_v[pl.ds(j * 16, 16)]
                sids = plsc.load_gather(srow, [eidx])
                dids = plsc.load_gather(drow, [eidx])
                a = plsc.load_gather(suv_v, [sids * eight + col])
                b = plsc.load_gather(suv_v, [dids * eight + col4])
                outv[pl.ds(j * 16, 16)] = a + b
            ooff = pl.multiple_of(t * 512, 512)
            pltpu.sync_copy(outv, out_h.at[pl.ds(ooff, 512)])
        return carry

    lax.fori_loop(0, CHUNKS, chunk, 0)


_escore = functools.partial(
    pl.kernel, _escore_body,
    out_type=jax.ShapeDtypeStruct((E_EDGES * 4,), jnp.float32),
    mesh=_MESH,
    compiler_params=pltpu.CompilerParams(needs_layout_passes=False),
    scratch_types=(
        pltpu.VMEM((N_NODES * 8,), jnp.float32),
        pltpu.VMEM((128,), jnp.int32),
        pltpu.VMEM((128,), jnp.int32),
        pltpu.VMEM((512,), jnp.float32),
        pltpu.VMEM((35 * 16,), jnp.int32),
    ),
)()


BLK = 400


def _tc1_body(x_ref, acc_ref, cnt_ref, ws_ref, wn_ref, b_ref, o_ref):
    agg = acc_ref[0] + acc_ref[1]
    cnt = cnt_ref[0, :, 0:1] + cnt_ref[1, :, 0:1]
    mean = agg / jnp.maximum(cnt, 1.0)
    h = (jnp.dot(x_ref[...], ws_ref[...], preferred_element_type=jnp.float32)
         + jnp.dot(mean, wn_ref[...], preferred_element_type=jnp.float32)
         + b_ref[...])
    o_ref[...] = jnp.maximum(h, 0.0)


def _tc2_body(h_ref, acc_ref, cnt_ref, ws_ref, wn_ref, b_ref, wc_ref, b8_ref,
              o_ref):
    agg = acc_ref[0] + acc_ref[1]
    cnt = cnt_ref[0, :, 0:1] + cnt_ref[1, :, 0:1]
    mean = agg / jnp.maximum(cnt, 1.0)
    h2 = (jnp.dot(h_ref[...], ws_ref[...], preferred_element_type=jnp.float32)
          + jnp.dot(mean, wn_ref[...], preferred_element_type=jnp.float32)
          + b_ref[...])
    o_ref[...] = (jnp.dot(h2, wc_ref[...], preferred_element_type=jnp.float32)
                  + b8_ref[...])


def _tc1(x, acc, cnt, ws, wn, b):
    return pl.pallas_call(
        _tc1_body,
        grid=(N_NODES // BLK,),
        in_specs=[
            pl.BlockSpec((BLK, FDIM), lambda i: (i, 0)),
            pl.BlockSpec((NC, BLK, FDIM), lambda i: (0, i, 0)),
            pl.BlockSpec((NC, BLK, FDIM), lambda i: (0, i, 0)),
            pl.BlockSpec((FDIM, FDIM), lambda i: (0, 0)),
            pl.BlockSpec((FDIM, FDIM), lambda i: (0, 0)),
            pl.BlockSpec((1, FDIM), lambda i: (0, 0)),
        ],
        out_specs=pl.BlockSpec((BLK, FDIM), lambda i: (i, 0)),
        out_shape=jax.ShapeDtypeStruct((N_NODES, FDIM), jnp.float32),
    )(x, acc, cnt, ws, wn, b)


def _tc2(h, acc, cnt, ws, wn, b, wc, b8):
    return pl.pallas_call(
        _tc2_body,
        grid=(N_NODES // BLK,),
        in_specs=[
            pl.BlockSpec((BLK, FDIM), lambda i: (i, 0)),
            pl.BlockSpec((NC, BLK, FDIM), lambda i: (0, i, 0)),
            pl.BlockSpec((NC, BLK, FDIM), lambda i: (0, i, 0)),
            pl.BlockSpec((FDIM, FDIM), lambda i: (0, 0)),
            pl.BlockSpec((FDIM, FDIM), lambda i: (0, 0)),
            pl.BlockSpec((1, FDIM), lambda i: (0, 0)),
            pl.BlockSpec((FDIM, 8), lambda i: (0, 0)),
            pl.BlockSpec((1, 8), lambda i: (0, 0)),
        ],
        out_specs=pl.BlockSpec((BLK, 8), lambda i: (i, 0)),
        out_shape=jax.ShapeDtypeStruct((N_NODES, 8), jnp.float32),
    )(h, acc, cnt, ws, wn, b, wc, b8)


def kernel(x, edge_index, e, W1_self, W1_neigh, b1, W2_self, W2_neigh, b2,
           Wp, bp):
    del e
    src1 = edge_index[0]
    dst1 = edge_index[1]
    z128 = jnp.zeros((ZCH, FDIM), jnp.float32)
    ones128 = jnp.ones((128, FDIM), jnp.float32)
    idxtab = jnp.asarray(
        [j * 4 + l // 4 for j in range(32) for l in range(16)]
        + [l % 4 for l in range(16)]
        + [4 + l % 4 for l in range(16)]
        + [8] * 16, jnp.int32)

    cnt = _cnt_kernel(dst1, z128, ones128).reshape(NC, N_NODES, FDIM)
    acc1 = _segsum(x, src1, dst1, z128).reshape(NC, N_NODES, FDIM)
    h1 = _tc1(x, acc1, cnt, W1_self, W1_neigh, b1.reshape(1, FDIM))
    acc2 = _segsum(h1, src1, dst1, z128).reshape(NC, N_NODES, FDIM)

    wu = jnp.pad(Wp[:FDIM], ((0, 0), (0, 1)))
    wv = jnp.pad(Wp[FDIM:], ((0, 0), (0, 1)))
    wc = jnp.concatenate([wu, wv], axis=1)
    b8 = jnp.concatenate([jnp.zeros((4,), jnp.float32), bp,
                          jnp.zeros((1,), jnp.float32)]).reshape(1, 8)
    suv = _tc2(h1, acc2, cnt, W2_self, W2_neigh, b2.reshape(1, FDIM), wc, b8)

    out4 = _escore(suv.reshape(-1), src1, dst1, idxtab)
    return out4.reshape(E_EDGES, 4)[:, :3]


# speedup vs baseline: 6.3273x; 1.1238x over previous
import functools

import jax
import jax.numpy as jnp
from jax import lax
from jax.experimental import pallas as pl
from jax.experimental.pallas import tpu as pltpu
from jax.experimental.pallas import tpu_sc as plsc

N_NODES = 10000
E_EDGES = 320000
FDIM = 128
EROWS = E_EDGES // 128
NC, NS = 2, 16
NW = NC * NS
CHUNKS = -(-EROWS // NW)
RPT = 640
ZCH = 80

_MESH = plsc.VectorSubcoreMesh(
    core_axis_name="c", subcore_axis_name="s", num_cores=NC, num_subcores=NS)


def _node_slices(sid):
    r0 = sid * RPT
    nrows = jnp.where(sid == NS - 1, 400, RPT)
    return r0, nrows


NBUF = 2


def _segsum_body(table, src1, dst1, z128,
                 acc_out,
                 srow, drow, rows, zbuf, semi, semg, sems,
                 acc_sh):
    cid = lax.axis_index("c")
    sid = lax.axis_index("s")
    wid = sid * NC + cid
    r0, nrows = _node_slices(sid)

    pltpu.sync_copy(z128, zbuf)
    for i in range(RPT // ZCH):
        @pl.when(i * ZCH < nrows)
        def _():
            off = pl.multiple_of(r0 + i * ZCH, 8)
            pltpu.sync_copy(zbuf, acc_sh.at[pl.ds(off, ZCH)])
    plsc.subcore_barrier()

    def chunk4(kk, carry):
        base = wid + kk * (NBUF * NW)
        for q in range(NBUF):
            t = base + q * NW

            @pl.when(t < EROWS)
            def _(t=t, q=q):
                eoff = pl.multiple_of(t * 128, 128)
                pltpu.async_copy(src1.at[pl.ds(eoff, 128)], srow.at[q], semi)
                pltpu.async_copy(dst1.at[pl.ds(eoff, 128)], drow.at[q], semi)
        for q in range(NBUF):
            t = base + q * NW

            @pl.when(t < EROWS)
            def _(t=t, q=q):
                eoff = pl.multiple_of(t * 128, 128)
                pltpu.make_async_copy(
                    src1.at[pl.ds(eoff, 128)], srow.at[q], semi).wait()
                pltpu.make_async_copy(
                    dst1.at[pl.ds(eoff, 128)], drow.at[q], semi).wait()
                pltpu.async_copy(table.at[srow.at[q]],
                                 rows.at[pl.ds(q * 128, 128)], semg)
        for q in range(NBUF):
            t = base + q * NW

            @pl.when(t < EROWS)
            def _(t=t, q=q):
                pltpu.make_async_copy(
                    table.at[srow.at[q]],
                    rows.at[pl.ds(q * 128, 128)], semg).wait()
                pltpu.async_copy(rows.at[pl.ds(q * 128, 128)],
                                 acc_sh.at[drow.at[q]], sems, add=True)
        for q in range(NBUF):
            t = base + q * NW

            @pl.when(t < EROWS)
            def _(t=t, q=q):
                pltpu.make_async_copy(
                    rows.at[pl.ds(q * 128, 128)],
                    acc_sh.at[drow.at[q]], sems).wait()
        return carry

    lax.fori_loop(0, -(-CHUNKS // NBUF), chunk4, 0)
    plsc.subcore_barrier()

    for i in range(RPT // ZCH):
        @pl.when(i * ZCH < nrows)
        def _():
            off = pl.multiple_of(r0 + i * ZCH, 8)
            hoff = pl.multiple_of(cid * N_NODES + r0 + i * ZCH, 8)
            pltpu.sync_copy(acc_sh.at[pl.ds(off, ZCH)], zbuf)
            pltpu.sync_copy(zbuf, acc_out.at[pl.ds(hoff, ZCH)])


_segsum = functools.partial(
    pl.kernel, _segsum_body,
    out_type=jax.ShapeDtypeStruct((NC * N_NODES, FDIM), jnp.float32),
    mesh=_MESH,
    compiler_params=pltpu.CompilerParams(needs_layout_passes=False),
    scratch_types=(
        pltpu.VMEM((NBUF, 128), jnp.int32),
        pltpu.VMEM((NBUF, 128), jnp.int32),
        pltpu.VMEM((NBUF * 128, FDIM), jnp.float32),
        pltpu.VMEM((ZCH, FDIM), jnp.float32),
        pltpu.SemaphoreType.DMA,
        pltpu.SemaphoreType.DMA,
        pltpu.SemaphoreType.DMA,
        pltpu.VMEM_SHARED((N_NODES, FDIM), jnp.float32),
    ),
)()


EPW = E_EDGES // NW


def _cnt_body(dst1, zcnt, fones,
              cnt_out,
              drow, cnt_v, ones_v):
    cid = lax.axis_index("c")
    sid = lax.axis_index("s")
    wid = sid * NC + cid

    pltpu.sync_copy(zcnt, cnt_v)
    pltpu.sync_copy(fones, ones_v)
    eoff = pl.multiple_of(wid * EPW, 8)
    pltpu.sync_copy(dst1.at[pl.ds(eoff, EPW)], drow)
    ones = ones_v[...]
    for g in range(EPW // 16):
        ids = drow[pl.ds(g * 16, 16)]
        plsc.addupdate_scatter(cnt_v, [ids], ones)
    ooff = pl.multiple_of(wid * N_NODES, 8)
    pltpu.sync_copy(cnt_v, cnt_out.at[pl.ds(ooff, N_NODES)])


_cnt_kernel = functools.partial(
    pl.kernel, _cnt_body,
    out_type=jax.ShapeDtypeStruct((NW * N_NODES,), jnp.float32),
    mesh=_MESH,
    compiler_params=pltpu.CompilerParams(needs_layout_passes=False),
    scratch_types=(
        pltpu.VMEM((EPW,), jnp.int32),
        pltpu.VMEM((N_NODES,), jnp.float32),
        pltpu.VMEM((16,), jnp.float32),
    ),
)()


def _escore_body(suv_h, src1, dst1, idxtab_h,
                 out_h,
                 suv_v, srow, drow, outv, idx_v):
    cid = lax.axis_index("c")
    sid = lax.axis_index("s")
    wid = sid * NC + cid

    pltpu.sync_copy(suv_h, suv_v)
    pltpu.sync_copy(idxtab_h, idx_v)

    def chunk(k, carry):
        t = wid + k * NW

        @pl.when(t < EROWS)
        def _():
            eoff = pl.multiple_of(t * 128, 128)
            pltpu.sync_copy(src1.at[pl.ds(eoff, 128)], srow)
            pltpu.sync_copy(dst1.at[pl.ds(eoff, 128)], drow)
            col = idx_v[pl.ds(32 * 16, 16)]
            col4 = idx_v[pl.ds(33 * 16, 16)]
            eight = idx_v[pl.ds(34 * 16, 16)]
            for j in range(32):
                eidx = idx_v[pl.ds(j * 16, 16)]
                sids = plsc.load_gather(srow, [eidx])
                dids = plsc.load_gather(drow, [eidx])
                a = plsc.load_gather(suv_v, [sids * eight + col])
                b = plsc.load_gather(suv_v, [dids * eight + col4])
                outv[pl.ds(j * 16, 16)] = a + b
            ooff = pl.multiple_of(t * 512, 512)
            pltpu.sync_copy(outv, out_h.at[pl.ds(ooff, 512)])
        return carry

    lax.fori_loop(0, CHUNKS, chunk, 0)


_escore = functools.partial(
    pl.kernel, _escore_body,
    out_type=jax.ShapeDtypeStruct((E_EDGES * 4,), jnp.float32),
    mesh=_MESH,
    compiler_params=pltpu.CompilerParams(needs_layout_passes=False),
    scratch_types=(
        pltpu.VMEM((N_NODES * 8,), jnp.float32),
        pltpu.VMEM((128,), jnp.int32),
        pltpu.VMEM((128,), jnp.int32),
        pltpu.VMEM((512,), jnp.float32),
        pltpu.VMEM((35 * 16,), jnp.int32),
    ),
)()


BLK = 400


def _tc1_body(x_ref, acc_ref, cnt_ref, ws_ref, wn_ref, b_ref, o_ref):
    agg = acc_ref[0] + acc_ref[1]
    cnt = jnp.sum(cnt_ref[...], axis=1)[:, None]
    mean = agg / jnp.maximum(cnt, 1.0)
    h = (jnp.dot(x_ref[...], ws_ref[...], preferred_element_type=jnp.float32)
         + jnp.dot(mean, wn_ref[...], preferred_element_type=jnp.float32)
         + b_ref[...])
    o_ref[...] = jnp.maximum(h, 0.0)


def _tc2_body(h_ref, acc_ref, cnt_ref, ws_ref, wn_ref, b_ref, wc_ref, b8_ref,
              o_ref):
    agg = acc_ref[0] + acc_ref[1]
    cnt = jnp.sum(cnt_ref[...], axis=1)[:, None]
    mean = agg / jnp.maximum(cnt, 1.0)
    h2 = (jnp.dot(h_ref[...], ws_ref[...], preferred_element_type=jnp.float32)
          + jnp.dot(mean, wn_ref[...], preferred_element_type=jnp.float32)
          + b_ref[...])
    o_ref[...] = (jnp.dot(h2, wc_ref[...], preferred_element_type=jnp.float32)
                  + b8_ref[...])


def _tc1(x, acc, cnt, ws, wn, b):
    return pl.pallas_call(
        _tc1_body,
        grid=(N_NODES // BLK,),
        in_specs=[
            pl.BlockSpec((BLK, FDIM), lambda i: (i, 0)),
            pl.BlockSpec((NC, BLK, FDIM), lambda i: (0, i, 0)),
            pl.BlockSpec((BLK, NW), lambda i: (i, 0)),
            pl.BlockSpec((FDIM, FDIM), lambda i: (0, 0)),
            pl.BlockSpec((FDIM, FDIM), lambda i: (0, 0)),
            pl.BlockSpec((1, FDIM), lambda i: (0, 0)),
        ],
        out_specs=pl.BlockSpec((BLK, FDIM), lambda i: (i, 0)),
        out_shape=jax.ShapeDtypeStruct((N_NODES, FDIM), jnp.float32),
    )(x, acc, cnt, ws, wn, b)


def _tc2(h, acc, cnt, ws, wn, b, wc, b8):
    return pl.pallas_call(
        _tc2_body,
        grid=(N_NODES // BLK,),
        in_specs=[
            pl.BlockSpec((BLK, FDIM), lambda i: (i, 0)),
            pl.BlockSpec((NC, BLK, FDIM), lambda i: (0, i, 0)),
            pl.BlockSpec((BLK, NW), lambda i: (i, 0)),
            pl.BlockSpec((FDIM, FDIM), lambda i: (0, 0)),
            pl.BlockSpec((FDIM, FDIM), lambda i: (0, 0)),
            pl.BlockSpec((1, FDIM), lambda i: (0, 0)),
            pl.BlockSpec((FDIM, 8), lambda i: (0, 0)),
            pl.BlockSpec((1, 8), lambda i: (0, 0)),
        ],
        out_specs=pl.BlockSpec((BLK, 8), lambda i: (i, 0)),
        out_shape=jax.ShapeDtypeStruct((N_NODES, 8), jnp.float32),
    )(h, acc, cnt, ws, wn, b, wc, b8)


def kernel(x, edge_index, e, W1_self, W1_neigh, b1, W2_self, W2_neigh, b2,
           Wp, bp):
    del e
    src1 = edge_index[0]
    dst1 = edge_index[1]
    z128 = jnp.zeros((ZCH, FDIM), jnp.float32)
    zcnt = jnp.zeros((N_NODES,), jnp.float32)
    fones = jnp.ones((16,), jnp.float32)
    idxtab = jnp.asarray(
        [j * 4 + l // 4 for j in range(32) for l in range(16)]
        + [l % 4 for l in range(16)]
        + [4 + l % 4 for l in range(16)]
        + [8] * 16, jnp.int32)

    cnt = _cnt_kernel(dst1, zcnt, fones).reshape(NW, N_NODES).T
    acc1 = _segsum(x, src1, dst1, z128).reshape(NC, N_NODES, FDIM)
    h1 = _tc1(x, acc1, cnt, W1_self, W1_neigh, b1.reshape(1, FDIM))
    acc2 = _segsum(h1, src1, dst1, z128).reshape(NC, N_NODES, FDIM)

    wu = jnp.pad(Wp[:FDIM], ((0, 0), (0, 1)))
    wv = jnp.pad(Wp[FDIM:], ((0, 0), (0, 1)))
    wc = jnp.concatenate([wu, wv], axis=1)
    b8 = jnp.concatenate([jnp.zeros((4,), jnp.float32), bp,
                          jnp.zeros((1,), jnp.float32)]).reshape(1, 8)
    suv = _tc2(h1, acc2, cnt, W2_self, W2_neigh, b2.reshape(1, FDIM), wc, b8)

    out4 = _escore(suv.reshape(-1), src1, dst1, idxtab)
    return out4.reshape(E_EDGES, 4)[:, :3]


# speedup vs baseline: 6.5037x; 1.0279x over previous
import functools

import jax
import jax.numpy as jnp
from jax import lax
from jax.experimental import pallas as pl
from jax.experimental.pallas import tpu as pltpu
from jax.experimental.pallas import tpu_sc as plsc

N_NODES = 10000
E_EDGES = 320000
FDIM = 128
EROWS = E_EDGES // 128
NC, NS = 2, 16
NW = NC * NS
CHUNKS = -(-EROWS // NW)
RPT = 640
ZCH = 80

_MESH = plsc.VectorSubcoreMesh(
    core_axis_name="c", subcore_axis_name="s", num_cores=NC, num_subcores=NS)


def _node_slices(sid):
    r0 = sid * RPT
    nrows = jnp.where(sid == NS - 1, 400, RPT)
    return r0, nrows


NBUF = 2


def _segsum_body(table, src1, dst1, z128,
                 acc_out,
                 srow, drow, rows, zbuf, semi, semg, sems,
                 acc_sh):
    cid = lax.axis_index("c")
    sid = lax.axis_index("s")
    wid = sid * NC + cid
    r0, nrows = _node_slices(sid)

    pltpu.sync_copy(z128, zbuf)
    for i in range(RPT // ZCH):
        @pl.when(i * ZCH < nrows)
        def _():
            off = pl.multiple_of(r0 + i * ZCH, 8)
            pltpu.sync_copy(zbuf, acc_sh.at[pl.ds(off, ZCH)])
    plsc.subcore_barrier()

    def chunk4(kk, carry):
        base = wid + kk * (NBUF * NW)
        for q in range(NBUF):
            t = base + q * NW

            @pl.when(t < EROWS)
            def _(t=t, q=q):
                eoff = pl.multiple_of(t * 128, 128)
                pltpu.async_copy(src1.at[pl.ds(eoff, 128)], srow.at[q], semi)
                pltpu.async_copy(dst1.at[pl.ds(eoff, 128)], drow.at[q], semi)
        for q in range(NBUF):
            t = base + q * NW

            @pl.when(t < EROWS)
            def _(t=t, q=q):
                eoff = pl.multiple_of(t * 128, 128)
                pltpu.make_async_copy(
                    src1.at[pl.ds(eoff, 128)], srow.at[q], semi).wait()
                pltpu.make_async_copy(
                    dst1.at[pl.ds(eoff, 128)], drow.at[q], semi).wait()
                pltpu.async_copy(table.at[srow.at[q]],
                                 rows.at[pl.ds(q * 128, 128)], semg)
        for q in range(NBUF):
            t = base + q * NW

            @pl.when(t < EROWS)
            def _(t=t, q=q):
                pltpu.make_async_copy(
                    table.at[srow.at[q]],
                    rows.at[pl.ds(q * 128, 128)], semg).wait()
                pltpu.async_copy(rows.at[pl.ds(q * 128, 128)],
                                 acc_sh.at[drow.at[q]], sems, add=True)
        for q in range(NBUF):
            t = base + q * NW

            @pl.when(t < EROWS)
            def _(t=t, q=q):
                pltpu.make_async_copy(
                    rows.at[pl.ds(q * 128, 128)],
                    acc_sh.at[drow.at[q]], sems).wait()
        return carry

    lax.fori_loop(0, -(-CHUNKS // NBUF), chunk4, 0)
    plsc.subcore_barrier()

    for i in range(RPT // ZCH):
        @pl.when(i * ZCH < nrows)
        def _():
            off = pl.multiple_of(r0 + i * ZCH, 8)
            hoff = pl.multiple_of(cid * N_NODES + r0 + i * ZCH, 8)
            pltpu.sync_copy(acc_sh.at[pl.ds(off, ZCH)], zbuf)
            pltpu.sync_copy(zbuf, acc_out.at[pl.ds(hoff, ZCH)])


_segsum = functools.partial(
    pl.kernel, _segsum_body,
    out_type=jax.ShapeDtypeStruct((NC * N_NODES, FDIM), jnp.float32),
    mesh=_MESH,
    compiler_params=pltpu.CompilerParams(needs_layout_passes=False),
    scratch_types=(
        pltpu.VMEM((NBUF, 128), jnp.int32),
        pltpu.VMEM((NBUF, 128), jnp.int32),
        pltpu.VMEM((NBUF * 128, FDIM), jnp.float32),
        pltpu.VMEM((ZCH, FDIM), jnp.float32),
        pltpu.SemaphoreType.DMA,
        pltpu.SemaphoreType.DMA,
        pltpu.SemaphoreType.DMA,
        pltpu.VMEM_SHARED((N_NODES, FDIM), jnp.float32),
    ),
)()


EPW = E_EDGES // NW


def _cnt_body(dst1, zcnt, fones,
              cnt_out,
              drow, cnt_v, ones_v):
    cid = lax.axis_index("c")
    sid = lax.axis_index("s")
    wid = sid * NC + cid

    pltpu.sync_copy(zcnt, cnt_v)
    pltpu.sync_copy(fones, ones_v)
    eoff = pl.multiple_of(wid * EPW, 8)
    pltpu.sync_copy(dst1.at[pl.ds(eoff, EPW)], drow)
    ones = ones_v[...]
    for g in range(EPW // 16):
        ids = drow[pl.ds(g * 16, 16)]
        plsc.addupdate_scatter(cnt_v, [ids], ones)
    ooff = pl.multiple_of(wid * N_NODES, 8)
    pltpu.sync_copy(cnt_v, cnt_out.at[pl.ds(ooff, N_NODES)])


_cnt_kernel = functools.partial(
    pl.kernel, _cnt_body,
    out_type=jax.ShapeDtypeStruct((NW * N_NODES,), jnp.float32),
    mesh=_MESH,
    compiler_params=pltpu.CompilerParams(needs_layout_passes=False),
    scratch_types=(
        pltpu.VMEM((EPW,), jnp.int32),
        pltpu.VMEM((N_NODES,), jnp.float32),
        pltpu.VMEM((16,), jnp.float32),
    ),
)()


def _escore_body(suv_h, src1, dst1, idxtab_h,
                 out_h,
                 suv_v, srow, drow, outv, idx_v):
    cid = lax.axis_index("c")
    sid = lax.axis_index("s")
    wid = sid * NC + cid

    pltpu.sync_copy(suv_h, suv_v)
    pltpu.sync_copy(idxtab_h, idx_v)

    def chunk(k, carry):
        t = wid + k * NW

        @pl.when(t < EROWS)
        def _():
            eoff = pl.multiple_of(t * 128, 128)
            pltpu.sync_copy(src1.at[pl.ds(eoff, 128)], srow)
            pltpu.sync_copy(dst1.at[pl.ds(eoff, 128)], drow)
            eight = idx_v[pl.ds(30 * 16, 16)]
            for g in range(24):
                eidx = idx_v[pl.ds(g * 16, 16)]
                colu = idx_v[pl.ds((24 + g % 3) * 16, 16)]
                colv = idx_v[pl.ds((27 + g % 3) * 16, 16)]
                sids = plsc.load_gather(srow, [eidx])
                dids = plsc.load_gather(drow, [eidx])
                a = plsc.load_gather(suv_v, [sids * eight + colu])
                b = plsc.load_gather(suv_v, [dids * eight + colv])
                outv[pl.ds(g * 16, 16)] = a + b
            ooff = pl.multiple_of(t * 384, 8)
            pltpu.sync_copy(outv, out_h.at[pl.ds(ooff, 384)])
        return carry

    lax.fori_loop(0, CHUNKS, chunk, 0)


_escore = functools.partial(
    pl.kernel, _escore_body,
    out_type=jax.ShapeDtypeStruct((E_EDGES * 3,), jnp.float32),
    mesh=_MESH,
    compiler_params=pltpu.CompilerParams(needs_layout_passes=False),
    scratch_types=(
        pltpu.VMEM((N_NODES * 8,), jnp.float32),
        pltpu.VMEM((128,), jnp.int32),
        pltpu.VMEM((128,), jnp.int32),
        pltpu.VMEM((384,), jnp.float32),
        pltpu.VMEM((31 * 16,), jnp.int32),
    ),
)()


BLK = 400


def _tc1_body(x_ref, acc_ref, cnt_ref, ws_ref, wn_ref, b_ref, o_ref):
    agg = acc_ref[0] + acc_ref[1]
    cnt = jnp.sum(cnt_ref[...], axis=1)[:, None]
    mean = agg / jnp.maximum(cnt, 1.0)
    h = (jnp.dot(x_ref[...], ws_ref[...], preferred_element_type=jnp.float32)
         + jnp.dot(mean, wn_ref[...], preferred_element_type=jnp.float32)
         + b_ref[...])
    o_ref[...] = jnp.maximum(h, 0.0)


def _tc2_body(h_ref, acc_ref, cnt_ref, ws_ref, wn_ref, b_ref, wc_ref, b8_ref,
              o_ref):
    agg = acc_ref[0] + acc_ref[1]
    cnt = jnp.sum(cnt_ref[...], axis=1)[:, None]
    mean = agg / jnp.maximum(cnt, 1.0)
    h2 = (jnp.dot(h_ref[...], ws_ref[...], preferred_element_type=jnp.float32)
          + jnp.dot(mean, wn_ref[...], preferred_element_type=jnp.float32)
          + b_ref[...])
    o_ref[...] = (jnp.dot(h2, wc_ref[...], preferred_element_type=jnp.float32)
                  + b8_ref[...])


def _tc1(x, acc, cnt, ws, wn, b):
    return pl.pallas_call(
        _tc1_body,
        grid=(N_NODES // BLK,),
        in_specs=[
            pl.BlockSpec((BLK, FDIM), lambda i: (i, 0)),
            pl.BlockSpec((NC, BLK, FDIM), lambda i: (0, i, 0)),
            pl.BlockSpec((BLK, NW), lambda i: (i, 0)),
            pl.BlockSpec((FDIM, FDIM), lambda i: (0, 0)),
            pl.BlockSpec((FDIM, FDIM), lambda i: (0, 0)),
            pl.BlockSpec((1, FDIM), lambda i: (0, 0)),
        ],
        out_specs=pl.BlockSpec((BLK, FDIM), lambda i: (i, 0)),
        out_shape=jax.ShapeDtypeStruct((N_NODES, FDIM), jnp.float32),
    )(x, acc, cnt, ws, wn, b)


def _tc2(h, acc, cnt, ws, wn, b, wc, b8):
    return pl.pallas_call(
        _tc2_body,
        grid=(N_NODES // BLK,),
        in_specs=[
            pl.BlockSpec((BLK, FDIM), lambda i: (i, 0)),
            pl.BlockSpec((NC, BLK, FDIM), lambda i: (0, i, 0)),
            pl.BlockSpec((BLK, NW), lambda i: (i, 0)),
            pl.BlockSpec((FDIM, FDIM), lambda i: (0, 0)),
            pl.BlockSpec((FDIM, FDIM), lambda i: (0, 0)),
            pl.BlockSpec((1, FDIM), lambda i: (0, 0)),
            pl.BlockSpec((FDIM, 8), lambda i: (0, 0)),
            pl.BlockSpec((1, 8), lambda i: (0, 0)),
        ],
        out_specs=pl.BlockSpec((BLK, 8), lambda i: (i, 0)),
        out_shape=jax.ShapeDtypeStruct((N_NODES, 8), jnp.float32),
    )(h, acc, cnt, ws, wn, b, wc, b8)


def kernel(x, edge_index, e, W1_self, W1_neigh, b1, W2_self, W2_neigh, b2,
           Wp, bp):
    del e
    src1 = edge_index[0]
    dst1 = edge_index[1]
    z128 = jnp.zeros((ZCH, FDIM), jnp.float32)
    zcnt = jnp.zeros((N_NODES,), jnp.float32)
    fones = jnp.ones((16,), jnp.float32)
    idxtab = jnp.asarray(
        [(g * 16 + l) // 3 for g in range(24) for l in range(16)]
        + [(r * 16 + l) % 3 for r in range(3) for l in range(16)]
        + [4 + (r * 16 + l) % 3 for r in range(3) for l in range(16)]
        + [8] * 16, jnp.int32)

    cnt = _cnt_kernel(dst1, zcnt, fones).reshape(NW, N_NODES).T
    acc1 = _segsum(x, src1, dst1, z128).reshape(NC, N_NODES, FDIM)
    h1 = _tc1(x, acc1, cnt, W1_self, W1_neigh, b1.reshape(1, FDIM))
    acc2 = _segsum(h1, src1, dst1, z128).reshape(NC, N_NODES, FDIM)

    wu = jnp.pad(Wp[:FDIM], ((0, 0), (0, 1)))
    wv = jnp.pad(Wp[FDIM:], ((0, 0), (0, 1)))
    wc = jnp.concatenate([wu, wv], axis=1)
    b8 = jnp.concatenate([jnp.zeros((4,), jnp.float32), bp,
                          jnp.zeros((1,), jnp.float32)]).reshape(1, 8)
    suv = _tc2(h1, acc2, cnt, W2_self, W2_neigh, b2.reshape(1, FDIM), wc, b8)

    out3 = _escore(suv.reshape(-1), src1, dst1, idxtab)
    return out3.reshape(E_EDGES, 3)


# speedup vs baseline: 7.3868x; 1.1358x over previous
import functools

import jax
import jax.numpy as jnp
from jax import lax
from jax.experimental import pallas as pl
from jax.experimental.pallas import tpu as pltpu
from jax.experimental.pallas import tpu_sc as plsc

N_NODES = 10000
E_EDGES = 320000
FDIM = 128
EROWS = E_EDGES // 128
NC, NS = 2, 16
NW = NC * NS
CHUNKS = -(-EROWS // NW)
RPT = 640
ZCH = 80

_MESH = plsc.VectorSubcoreMesh(
    core_axis_name="c", subcore_axis_name="s", num_cores=NC, num_subcores=NS)


def _node_slices(sid):
    r0 = sid * RPT
    nrows = jnp.where(sid == NS - 1, 400, RPT)
    return r0, nrows


NBUF = 2


def _segsum_body(table, src1, dst1, z128,
                 acc_out,
                 srow, drow, rows, zbuf, semi, semg, sems,
                 acc_sh):
    cid = lax.axis_index("c")
    sid = lax.axis_index("s")
    wid = sid * NC + cid
    r0, nrows = _node_slices(sid)

    pltpu.sync_copy(z128, zbuf)
    for i in range(RPT // ZCH):
        @pl.when(i * ZCH < nrows)
        def _():
            off = pl.multiple_of(r0 + i * ZCH, 8)
            pltpu.sync_copy(zbuf, acc_sh.at[pl.ds(off, ZCH)])
    plsc.subcore_barrier()

    def chunk4(kk, carry):
        base = wid + kk * (NBUF * NW)
        for q in range(NBUF):
            t = base + q * NW

            @pl.when(t < EROWS)
            def _(t=t, q=q):
                eoff = pl.multiple_of(t * 128, 128)
                pltpu.async_copy(src1.at[pl.ds(eoff, 128)], srow.at[q], semi)
                pltpu.async_copy(dst1.at[pl.ds(eoff, 128)], drow.at[q], semi)
        for q in range(NBUF):
            t = base + q * NW

            @pl.when(t < EROWS)
            def _(t=t, q=q):
                eoff = pl.multiple_of(t * 128, 128)
                pltpu.make_async_copy(
                    src1.at[pl.ds(eoff, 128)], srow.at[q], semi).wait()
                pltpu.make_async_copy(
                    dst1.at[pl.ds(eoff, 128)], drow.at[q], semi).wait()
                pltpu.async_copy(table.at[srow.at[q]],
                                 rows.at[pl.ds(q * 128, 128)], semg)
        for q in range(NBUF):
            t = base + q * NW

            @pl.when(t < EROWS)
            def _(t=t, q=q):
                pltpu.make_async_copy(
                    table.at[srow.at[q]],
                    rows.at[pl.ds(q * 128, 128)], semg).wait()
                pltpu.async_copy(rows.at[pl.ds(q * 128, 128)],
                                 acc_sh.at[drow.at[q]], sems, add=True)
        for q in range(NBUF):
            t = base + q * NW

            @pl.when(t < EROWS)
            def _(t=t, q=q):
                pltpu.make_async_copy(
                    rows.at[pl.ds(q * 128, 128)],
                    acc_sh.at[drow.at[q]], sems).wait()
        return carry

    lax.fori_loop(0, -(-CHUNKS // NBUF), chunk4, 0)
    plsc.subcore_barrier()

    for i in range(RPT // ZCH):
        @pl.when(i * ZCH < nrows)
        def _():
            off = pl.multiple_of(r0 + i * ZCH, 8)
            hoff = pl.multiple_of(cid * N_NODES + r0 + i * ZCH, 8)
            pltpu.sync_copy(acc_sh.at[pl.ds(off, ZCH)], zbuf)
            pltpu.sync_copy(zbuf, acc_out.at[pl.ds(hoff, ZCH)])


_segsum = functools.partial(
    pl.kernel, _segsum_body,
    out_type=jax.ShapeDtypeStruct((NC * N_NODES, FDIM), jnp.float32),
    mesh=_MESH,
    compiler_params=pltpu.CompilerParams(needs_layout_passes=False),
    scratch_types=(
        pltpu.VMEM((NBUF, 128), jnp.int32),
        pltpu.VMEM((NBUF, 128), jnp.int32),
        pltpu.VMEM((NBUF * 128, FDIM), jnp.float32),
        pltpu.VMEM((ZCH, FDIM), jnp.float32),
        pltpu.SemaphoreType.DMA,
        pltpu.SemaphoreType.DMA,
        pltpu.SemaphoreType.DMA,
        pltpu.VMEM_SHARED((N_NODES, FDIM), jnp.float32),
    ),
)()


EPW = E_EDGES // NW


def _cnt_body(dst1, zcnt, fones,
              cnt_out,
              drow, cnt_v, ones_v):
    cid = lax.axis_index("c")
    sid = lax.axis_index("s")
    wid = sid * NC + cid

    pltpu.sync_copy(zcnt, cnt_v)
    pltpu.sync_copy(fones, ones_v)
    eoff = pl.multiple_of(wid * EPW, 8)
    pltpu.sync_copy(dst1.at[pl.ds(eoff, EPW)], drow)
    ones = ones_v[...]
    for g in range(EPW // 16):
        ids = drow[pl.ds(g * 16, 16)]
        plsc.addupdate_scatter(cnt_v, [ids], ones)
    ooff = pl.multiple_of(wid * N_NODES, 8)
    pltpu.sync_copy(cnt_v, cnt_out.at[pl.ds(ooff, N_NODES)])


_cnt_kernel = functools.partial(
    pl.kernel, _cnt_body,
    out_type=jax.ShapeDtypeStruct((NW * N_NODES,), jnp.float32),
    mesh=_MESH,
    compiler_params=pltpu.CompilerParams(needs_layout_passes=False),
    scratch_types=(
        pltpu.VMEM((EPW,), jnp.int32),
        pltpu.VMEM((N_NODES,), jnp.float32),
        pltpu.VMEM((16,), jnp.float32),
    ),
)()


def _escore_body(suv_h, src1, dst1, idxtab_h, zrow_h,
                 out_h,
                 suv_v, srow, drow, outv, idx_v, zrow_v):
    cid = lax.axis_index("c")
    sid = lax.axis_index("s")
    wid = sid * NC + cid

    pltpu.sync_copy(suv_h, suv_v)
    pltpu.sync_copy(idxtab_h, idx_v)
    pltpu.sync_copy(zrow_h, zrow_v)
    zv = zrow_v[...]
    for r in range(3, 8):
        for g in range(8):
            outv[r, pl.ds(g * 16, 16)] = zv

    def chunk(k, carry):
        t = wid + k * NW

        @pl.when(t < EROWS)
        def _():
            eoff = pl.multiple_of(t * 128, 128)
            pltpu.sync_copy(src1.at[pl.ds(eoff, 128)], srow)
            pltpu.sync_copy(dst1.at[pl.ds(eoff, 128)], drow)
            eight = idx_v[pl.ds(6 * 16, 16)]
            for g in range(8):
                sids = srow[pl.ds(g * 16, 16)]
                dids = drow[pl.ds(g * 16, 16)]
                s8 = sids * eight
                d8 = dids * eight
                for c in range(3):
                    cu = idx_v[pl.ds(c * 16, 16)]
                    cv = idx_v[pl.ds((3 + c) * 16, 16)]
                    a = plsc.load_gather(suv_v, [s8 + cu])
                    b = plsc.load_gather(suv_v, [d8 + cv])
                    outv[c, pl.ds(g * 16, 16)] = a + b
            toff = pl.multiple_of(t * 128, 128)
            pltpu.sync_copy(outv, out_h.at[:, pl.ds(toff, 128)])
        return carry

    lax.fori_loop(0, CHUNKS, chunk, 0)


_escore = functools.partial(
    pl.kernel, _escore_body,
    out_type=jax.ShapeDtypeStruct((8, E_EDGES), jnp.float32),
    mesh=_MESH,
    compiler_params=pltpu.CompilerParams(needs_layout_passes=False),
    scratch_types=(
        pltpu.VMEM((N_NODES * 8,), jnp.float32),
        pltpu.VMEM((128,), jnp.int32),
        pltpu.VMEM((128,), jnp.int32),
        pltpu.VMEM((8, 128), jnp.float32),
        pltpu.VMEM((7 * 16,), jnp.int32),
        pltpu.VMEM((16,), jnp.float32),
    ),
)()


BLK = 400


def _tc1_body(x_ref, acc_ref, cnt_ref, ws_ref, wn_ref, b_ref, o_ref):
    agg = acc_ref[0] + acc_ref[1]
    cnt = jnp.sum(cnt_ref[...], axis=1)[:, None]
    mean = agg / jnp.maximum(cnt, 1.0)
    h = (jnp.dot(x_ref[...], ws_ref[...], preferred_element_type=jnp.float32)
         + jnp.dot(mean, wn_ref[...], preferred_element_type=jnp.float32)
         + b_ref[...])
    o_ref[...] = jnp.maximum(h, 0.0)


def _tc2_body(h_ref, acc_ref, cnt_ref, ws_ref, wn_ref, b_ref, wc_ref, b8_ref,
              o_ref):
    agg = acc_ref[0] + acc_ref[1]
    cnt = jnp.sum(cnt_ref[...], axis=1)[:, None]
    mean = agg / jnp.maximum(cnt, 1.0)
    h2 = (jnp.dot(h_ref[...], ws_ref[...], preferred_element_type=jnp.float32)
          + jnp.dot(mean, wn_ref[...], preferred_element_type=jnp.float32)
          + b_ref[...])
    o_ref[...] = (jnp.dot(h2, wc_ref[...], preferred_element_type=jnp.float32)
                  + b8_ref[...])


def _tc1(x, acc, cnt, ws, wn, b):
    return pl.pallas_call(
        _tc1_body,
        grid=(N_NODES // BLK,),
        in_specs=[
            pl.BlockSpec((BLK, FDIM), lambda i: (i, 0)),
            pl.BlockSpec((NC, BLK, FDIM), lambda i: (0, i, 0)),
            pl.BlockSpec((BLK, NW), lambda i: (i, 0)),
            pl.BlockSpec((FDIM, FDIM), lambda i: (0, 0)),
            pl.BlockSpec((FDIM, FDIM), lambda i: (0, 0)),
            pl.BlockSpec((1, FDIM), lambda i: (0, 0)),
        ],
        out_specs=pl.BlockSpec((BLK, FDIM), lambda i: (i, 0)),
        out_shape=jax.ShapeDtypeStruct((N_NODES, FDIM), jnp.float32),
    )(x, acc, cnt, ws, wn, b)


def _tc2(h, acc, cnt, ws, wn, b, wc, b8):
    return pl.pallas_call(
        _tc2_body,
        grid=(N_NODES // BLK,),
        in_specs=[
            pl.BlockSpec((BLK, FDIM), lambda i: (i, 0)),
            pl.BlockSpec((NC, BLK, FDIM), lambda i: (0, i, 0)),
            pl.BlockSpec((BLK, NW), lambda i: (i, 0)),
            pl.BlockSpec((FDIM, FDIM), lambda i: (0, 0)),
            pl.BlockSpec((FDIM, FDIM), lambda i: (0, 0)),
            pl.BlockSpec((1, FDIM), lambda i: (0, 0)),
            pl.BlockSpec((FDIM, 8), lambda i: (0, 0)),
            pl.BlockSpec((1, 8), lambda i: (0, 0)),
        ],
        out_specs=pl.BlockSpec((BLK, 8), lambda i: (i, 0)),
        out_shape=jax.ShapeDtypeStruct((N_NODES, 8), jnp.float32),
    )(h, acc, cnt, ws, wn, b, wc, b8)


FBLK = 6400


def _fin_body(c_ref, eye_ref, o_ref):
    o_ref[...] = jax.lax.dot_general(
        c_ref[...], eye_ref[...], (((0,), (0,)), ((), ())),
        preferred_element_type=jnp.float32)


def _fin(cols, eye83):
    return pl.pallas_call(
        _fin_body,
        grid=(E_EDGES // FBLK,),
        in_specs=[
            pl.BlockSpec((8, FBLK), lambda i: (0, i)),
            pl.BlockSpec((8, 3), lambda i: (0, 0)),
        ],
        out_specs=pl.BlockSpec((FBLK, 3), lambda i: (i, 0)),
        out_shape=jax.ShapeDtypeStruct((E_EDGES, 3), jnp.float32),
    )(cols, eye83)


def kernel(x, edge_index, e, W1_self, W1_neigh, b1, W2_self, W2_neigh, b2,
           Wp, bp):
    del e
    src1 = edge_index[0]
    dst1 = edge_index[1]
    z128 = jnp.zeros((ZCH, FDIM), jnp.float32)
    zcnt = jnp.zeros((N_NODES,), jnp.float32)
    fones = jnp.ones((16,), jnp.float32)
    idxtab = jnp.asarray(
        [c for c in range(3) for _ in range(16)]
        + [4 + c for c in range(3) for _ in range(16)]
        + [8] * 16, jnp.int32)

    cnt = _cnt_kernel(dst1, zcnt, fones).reshape(NW, N_NODES).T
    acc1 = _segsum(x, src1, dst1, z128).reshape(NC, N_NODES, FDIM)
    h1 = _tc1(x, acc1, cnt, W1_self, W1_neigh, b1.reshape(1, FDIM))
    acc2 = _segsum(h1, src1, dst1, z128).reshape(NC, N_NODES, FDIM)

    wu = jnp.pad(Wp[:FDIM], ((0, 0), (0, 1)))
    wv = jnp.pad(Wp[FDIM:], ((0, 0), (0, 1)))
    wc = jnp.concatenate([wu, wv], axis=1)
    b8 = jnp.concatenate([jnp.zeros((4,), jnp.float32), bp,
                          jnp.zeros((1,), jnp.float32)]).reshape(1, 8)
    suv = _tc2(h1, acc2, cnt, W2_self, W2_neigh, b2.reshape(1, FDIM), wc, b8)

    zrow = jnp.zeros((16,), jnp.float32)
    cols = _escore(suv.reshape(-1), src1, dst1, idxtab, zrow)
    return _fin(cols, jnp.eye(8, 3, dtype=jnp.float32))


# speedup vs baseline: 9.5331x; 1.2906x over previous
import functools

import jax
import jax.numpy as jnp
from jax import lax
from jax.experimental import pallas as pl
from jax.experimental.pallas import tpu as pltpu
from jax.experimental.pallas import tpu_sc as plsc

N_NODES = 10000
E_EDGES = 320000
FDIM = 128
EROWS = E_EDGES // 128
NC, NS = 2, 16
NW = NC * NS
CHUNKS = -(-EROWS // NW)
RPT = 640
ZCH = 80

_MESH = plsc.VectorSubcoreMesh(
    core_axis_name="c", subcore_axis_name="s", num_cores=NC, num_subcores=NS)


def _node_slices(sid):
    r0 = sid * RPT
    nrows = jnp.where(sid == NS - 1, 400, RPT)
    return r0, nrows


NBUF = 2


def _segsum_body(table, src1, dst1, z128,
                 acc_out,
                 srow, drow, rows, zbuf, semi, semg, sems,
                 acc_sh):
    cid = lax.axis_index("c")
    sid = lax.axis_index("s")
    wid = sid * NC + cid
    r0, nrows = _node_slices(sid)

    pltpu.sync_copy(z128, zbuf)
    for i in range(RPT // ZCH):
        @pl.when(i * ZCH < nrows)
        def _():
            off = pl.multiple_of(r0 + i * ZCH, 8)
            pltpu.sync_copy(zbuf, acc_sh.at[pl.ds(off, ZCH)])
    plsc.subcore_barrier()

    def chunk4(kk, carry):
        base = wid + kk * (NBUF * NW)
        for q in range(NBUF):
            t = base + q * NW

            @pl.when(t < EROWS)
            def _(t=t, q=q):
                eoff = pl.multiple_of(t * 128, 128)
                pltpu.async_copy(src1.at[pl.ds(eoff, 128)], srow.at[q], semi)
                pltpu.async_copy(dst1.at[pl.ds(eoff, 128)], drow.at[q], semi)
        for q in range(NBUF):
            t = base + q * NW

            @pl.when(t < EROWS)
            def _(t=t, q=q):
                eoff = pl.multiple_of(t * 128, 128)
                pltpu.make_async_copy(
                    src1.at[pl.ds(eoff, 128)], srow.at[q], semi).wait()
                pltpu.make_async_copy(
                    dst1.at[pl.ds(eoff, 128)], drow.at[q], semi).wait()
                pltpu.async_copy(table.at[srow.at[q]],
                                 rows.at[pl.ds(q * 128, 128)], semg)
        for q in range(NBUF):
            t = base + q * NW

            @pl.when(t < EROWS)
            def _(t=t, q=q):
                pltpu.make_async_copy(
                    table.at[srow.at[q]],
                    rows.at[pl.ds(q * 128, 128)], semg).wait()
                pltpu.async_copy(rows.at[pl.ds(q * 128, 128)],
                                 acc_sh.at[drow.at[q]], sems, add=True)
        for q in range(NBUF):
            t = base + q * NW

            @pl.when(t < EROWS)
            def _(t=t, q=q):
                pltpu.make_async_copy(
                    rows.at[pl.ds(q * 128, 128)],
                    acc_sh.at[drow.at[q]], sems).wait()
        return carry

    lax.fori_loop(0, -(-CHUNKS // NBUF), chunk4, 0)
    plsc.subcore_barrier()

    for i in range(RPT // ZCH):
        @pl.when(i * ZCH < nrows)
        def _():
            off = pl.multiple_of(r0 + i * ZCH, 8)
            hoff = pl.multiple_of(cid * N_NODES + r0 + i * ZCH, 8)
            pltpu.sync_copy(acc_sh.at[pl.ds(off, ZCH)], zbuf)
            pltpu.sync_copy(zbuf, acc_out.at[pl.ds(hoff, ZCH)])


_segsum = functools.partial(
    pl.kernel, _segsum_body,
    out_type=jax.ShapeDtypeStruct((NC * N_NODES, FDIM), jnp.float32),
    mesh=_MESH,
    compiler_params=pltpu.CompilerParams(needs_layout_passes=False),
    scratch_types=(
        pltpu.VMEM((NBUF, 128), jnp.int32),
        pltpu.VMEM((NBUF, 128), jnp.int32),
        pltpu.VMEM((NBUF * 128, FDIM), jnp.float32),
        pltpu.VMEM((ZCH, FDIM), jnp.float32),
        pltpu.SemaphoreType.DMA,
        pltpu.SemaphoreType.DMA,
        pltpu.SemaphoreType.DMA,
        pltpu.VMEM_SHARED((N_NODES, FDIM), jnp.float32),
    ),
)()


EPW = E_EDGES // NW


def _cnt_body(dst1, zcnt, fones,
              cnt_out,
              drow, cnt_v, ones_v):
    cid = lax.axis_index("c")
    sid = lax.axis_index("s")
    wid = sid * NC + cid

    pltpu.sync_copy(zcnt, cnt_v)
    pltpu.sync_copy(fones, ones_v)
    eoff = pl.multiple_of(wid * EPW, 8)
    pltpu.sync_copy(dst1.at[pl.ds(eoff, EPW)], drow)
    ones = ones_v[...]
    for g in range(EPW // 16):
        ids = drow[pl.ds(g * 16, 16)]
        plsc.addupdate_scatter(cnt_v, [ids], ones)
    ooff = pl.multiple_of(wid * N_NODES, 8)
    pltpu.sync_copy(cnt_v, cnt_out.at[pl.ds(ooff, N_NODES)])


_cnt_kernel = functools.partial(
    pl.kernel, _cnt_body,
    out_type=jax.ShapeDtypeStruct((NW * N_NODES,), jnp.float32),
    mesh=_MESH,
    compiler_params=pltpu.CompilerParams(needs_layout_passes=False),
    scratch_types=(
        pltpu.VMEM((EPW,), jnp.int32),
        pltpu.VMEM((N_NODES,), jnp.float32),
        pltpu.VMEM((16,), jnp.float32),
    ),
)()


def _escore_body(suv_h, src1, dst1, idxtab_h, zrow_h,
                 out_h,
                 suv_v, srow, drow, outv, idx_v, zrow_v):
    cid = lax.axis_index("c")
    sid = lax.axis_index("s")
    wid = sid * NC + cid

    pltpu.sync_copy(suv_h, suv_v)
    pltpu.sync_copy(idxtab_h, idx_v)
    pltpu.sync_copy(zrow_h, zrow_v)
    zv = zrow_v[...]
    for r in range(3, 8):
        for g in range(8):
            outv[r, pl.ds(g * 16, 16)] = zv

    def chunk(k, carry):
        t = wid + k * NW

        @pl.when(t < EROWS)
        def _():
            eoff = pl.multiple_of(t * 128, 128)
            pltpu.sync_copy(src1.at[pl.ds(eoff, 128)], srow)
            pltpu.sync_copy(dst1.at[pl.ds(eoff, 128)], drow)
            eight = idx_v[pl.ds(6 * 16, 16)]
            for g in range(8):
                sids = srow[pl.ds(g * 16, 16)]
                dids = drow[pl.ds(g * 16, 16)]
                s8 = sids * eight
                d8 = dids * eight
                for c in range(3):
                    cu = idx_v[pl.ds(c * 16, 16)]
                    cv = idx_v[pl.ds((3 + c) * 16, 16)]
                    a = plsc.load_gather(suv_v, [s8 + cu])
                    b = plsc.load_gather(suv_v, [d8 + cv])
                    outv[c, pl.ds(g * 16, 16)] = a + b
            toff = pl.multiple_of(t * 128, 128)
            pltpu.sync_copy(outv, out_h.at[:, pl.ds(toff, 128)])
        return carry

    lax.fori_loop(0, CHUNKS, chunk, 0)


_escore = functools.partial(
    pl.kernel, _escore_body,
    out_type=jax.ShapeDtypeStruct((8, E_EDGES), jnp.float32),
    mesh=_MESH,
    compiler_params=pltpu.CompilerParams(needs_layout_passes=False),
    scratch_types=(
        pltpu.VMEM((N_NODES * 8,), jnp.float32),
        pltpu.VMEM((128,), jnp.int32),
        pltpu.VMEM((128,), jnp.int32),
        pltpu.VMEM((8, 128), jnp.float32),
        pltpu.VMEM((7 * 16,), jnp.int32),
        pltpu.VMEM((16,), jnp.float32),
    ),
)()


BLK = 400


def _tc1_body(x_ref, acc_ref, cnt_ref, ws_ref, wn_ref, b_ref, o_ref):
    agg = acc_ref[0] + acc_ref[1]
    cnt = jnp.sum(cnt_ref[...], axis=1)[:, None]
    mean = agg / jnp.maximum(cnt, 1.0)
    h = (jnp.dot(x_ref[...], ws_ref[...], preferred_element_type=jnp.float32)
         + jnp.dot(mean, wn_ref[...], preferred_element_type=jnp.float32)
         + b_ref[...])
    o_ref[...] = jnp.maximum(h, 0.0)


def _tc2_body(h_ref, acc_ref, cnt_ref, ws_ref, wn_ref, b_ref, wc_ref, b8_ref,
              o_ref):
    agg = acc_ref[0] + acc_ref[1]
    cnt = jnp.sum(cnt_ref[...], axis=1)[:, None]
    mean = agg / jnp.maximum(cnt, 1.0)
    h2 = (jnp.dot(h_ref[...], ws_ref[...], preferred_element_type=jnp.float32)
          + jnp.dot(mean, wn_ref[...], preferred_element_type=jnp.float32)
          + b_ref[...])
    o_ref[...] = (jnp.dot(h2, wc_ref[...], preferred_element_type=jnp.float32)
                  + b8_ref[...])


def _tc1(x, acc, cnt, ws, wn, b):
    return pl.pallas_call(
        _tc1_body,
        grid=(N_NODES // BLK,),
        in_specs=[
            pl.BlockSpec((BLK, FDIM), lambda i: (i, 0)),
            pl.BlockSpec((NC, BLK, FDIM), lambda i: (0, i, 0)),
            pl.BlockSpec((BLK, NW), lambda i: (i, 0)),
            pl.BlockSpec((FDIM, FDIM), lambda i: (0, 0)),
            pl.BlockSpec((FDIM, FDIM), lambda i: (0, 0)),
            pl.BlockSpec((1, FDIM), lambda i: (0, 0)),
        ],
        out_specs=pl.BlockSpec((BLK, FDIM), lambda i: (i, 0)),
        out_shape=jax.ShapeDtypeStruct((N_NODES, FDIM), jnp.float32),
    )(x, acc, cnt, ws, wn, b)


def _tc2(h, acc, cnt, ws, wn, b, wc, b8):
    return pl.pallas_call(
        _tc2_body,
        grid=(N_NODES // BLK,),
        in_specs=[
            pl.BlockSpec((BLK, FDIM), lambda i: (i, 0)),
            pl.BlockSpec((NC, BLK, FDIM), lambda i: (0, i, 0)),
            pl.BlockSpec((BLK, NW), lambda i: (i, 0)),
            pl.BlockSpec((FDIM, FDIM), lambda i: (0, 0)),
            pl.BlockSpec((FDIM, FDIM), lambda i: (0, 0)),
            pl.BlockSpec((1, FDIM), lambda i: (0, 0)),
            pl.BlockSpec((FDIM, 8), lambda i: (0, 0)),
            pl.BlockSpec((1, 8), lambda i: (0, 0)),
        ],
        out_specs=pl.BlockSpec((BLK, 8), lambda i: (i, 0)),
        out_shape=jax.ShapeDtypeStruct((N_NODES, 8), jnp.float32),
    )(h, acc, cnt, ws, wn, b, wc, b8)


FBLK = 6400


def _fin_body(c_ref, eye_ref, o_ref):
    o_ref[...] = jax.lax.dot_general(
        c_ref[...], eye_ref[...], (((0,), (0,)), ((), ())),
        preferred_element_type=jnp.float32)


def _fin(cols, eye83):
    return pl.pallas_call(
        _fin_body,
        grid=(E_EDGES // FBLK,),
        in_specs=[
            pl.BlockSpec((8, FBLK), lambda i: (0, i)),
            pl.BlockSpec((8, 3), lambda i: (0, 0)),
        ],
        out_specs=pl.BlockSpec((FBLK, 3), lambda i: (i, 0)),
        out_shape=jax.ShapeDtypeStruct((E_EDGES, 3), jnp.float32),
    )(cols, eye83)


def kernel(x, edge_index, e, W1_self, W1_neigh, b1, W2_self, W2_neigh, b2,
           Wp, bp):
    del e
    src1 = edge_index[0]
    dst1 = edge_index[1]
    z128 = jnp.zeros((ZCH, FDIM), jnp.float32)
    zcnt = jnp.zeros((N_NODES,), jnp.float32)
    fones = jnp.ones((16,), jnp.float32)
    idxtab = jnp.asarray(
        [c for c in range(3) for _ in range(16)]
        + [4 + c for c in range(3) for _ in range(16)]
        + [8] * 16, jnp.int32)

    cnt = _cnt_kernel(dst1, zcnt, fones).reshape(NW, N_NODES).T
    acc1 = _segsum(x, src1, dst1, z128).reshape(NC, N_NODES, FDIM)
    h1 = _tc1(x, acc1, cnt, W1_self, W1_neigh, b1.reshape(1, FDIM))
    acc2 = _segsum(h1, src1, dst1, z128).reshape(NC, N_NODES, FDIM)

    wu = jnp.pad(Wp[:FDIM], ((0, 0), (0, 1)))
    wv = jnp.pad(Wp[FDIM:], ((0, 0), (0, 1)))
    wc = jnp.concatenate([wu, wv], axis=1)
    b8 = jnp.concatenate([jnp.zeros((4,), jnp.float32), bp,
                          jnp.zeros((1,), jnp.float32)]).reshape(1, 8)
    suv = _tc2(h1, acc2, cnt, W2_self, W2_neigh, b2.reshape(1, FDIM), wc, b8)

    zrow = jnp.zeros((16,), jnp.float32)
    cols = _escore(suv.reshape(-1), src1, dst1, idxtab, zrow)
    return cols[:3, :].T


# speedup vs baseline: 10.7388x; 1.1265x over previous
import functools

import jax
import jax.numpy as jnp
from jax import lax
from jax.experimental import pallas as pl
from jax.experimental.pallas import tpu as pltpu
from jax.experimental.pallas import tpu_sc as plsc

N_NODES = 10000
E_EDGES = 320000
FDIM = 128
EROWS = E_EDGES // 128
NC, NS = 2, 16
NW = NC * NS
CHUNKS = -(-EROWS // NW)
RPT = 640
ZCH = 80

_MESH = plsc.VectorSubcoreMesh(
    core_axis_name="c", subcore_axis_name="s", num_cores=NC, num_subcores=NS)


def _node_slices(sid):
    r0 = sid * RPT
    nrows = jnp.where(sid == NS - 1, 400, RPT)
    return r0, nrows


NBUF = 2


def _segsum_body(table, src1, dst1, z128,
                 acc_out,
                 srow, drow, rows, zbuf, semi, semg, sems,
                 acc_sh):
    cid = lax.axis_index("c")
    sid = lax.axis_index("s")
    wid = sid * NC + cid
    r0, nrows = _node_slices(sid)

    pltpu.sync_copy(z128, zbuf)
    for i in range(RPT // ZCH):
        @pl.when(i * ZCH < nrows)
        def _():
            off = pl.multiple_of(r0 + i * ZCH, 8)
            pltpu.sync_copy(zbuf, acc_sh.at[pl.ds(off, ZCH)])
    plsc.subcore_barrier()

    def chunk4(kk, carry):
        base = wid + kk * (NBUF * NW)
        for q in range(NBUF):
            t = base + q * NW

            @pl.when(t < EROWS)
            def _(t=t, q=q):
                eoff = pl.multiple_of(t * 128, 128)
                pltpu.async_copy(src1.at[pl.ds(eoff, 128)], srow.at[q], semi)
                pltpu.async_copy(dst1.at[pl.ds(eoff, 128)], drow.at[q], semi)
        for q in range(NBUF):
            t = base + q * NW

            @pl.when(t < EROWS)
            def _(t=t, q=q):
                eoff = pl.multiple_of(t * 128, 128)
                pltpu.make_async_copy(
                    src1.at[pl.ds(eoff, 128)], srow.at[q], semi).wait()
                pltpu.make_async_copy(
                    dst1.at[pl.ds(eoff, 128)], drow.at[q], semi).wait()
                pltpu.async_copy(table.at[srow.at[q]],
                                 rows.at[pl.ds(q * 128, 128)], semg)
        for q in range(NBUF):
            t = base + q * NW

            @pl.when(t < EROWS)
            def _(t=t, q=q):
                pltpu.make_async_copy(
                    table.at[srow.at[q]],
                    rows.at[pl.ds(q * 128, 128)], semg).wait()
                pltpu.async_copy(rows.at[pl.ds(q * 128, 128)],
                                 acc_sh.at[drow.at[q]], sems, add=True)
        for q in range(NBUF):
            t = base + q * NW

            @pl.when(t < EROWS)
            def _(t=t, q=q):
                pltpu.make_async_copy(
                    rows.at[pl.ds(q * 128, 128)],
                    acc_sh.at[drow.at[q]], sems).wait()
        return carry

    lax.fori_loop(0, -(-CHUNKS // NBUF), chunk4, 0)
    plsc.subcore_barrier()

    for i in range(RPT // ZCH):
        @pl.when(i * ZCH < nrows)
        def _():
            off = pl.multiple_of(r0 + i * ZCH, 8)
            hoff = pl.multiple_of(cid * N_NODES + r0 + i * ZCH, 8)
            pltpu.sync_copy(acc_sh.at[pl.ds(off, ZCH)], zbuf)
            pltpu.sync_copy(zbuf, acc_out.at[pl.ds(hoff, ZCH)])


_segsum = functools.partial(
    pl.kernel, _segsum_body,
    out_type=jax.ShapeDtypeStruct((NC * N_NODES, FDIM), jnp.float32),
    mesh=_MESH,
    compiler_params=pltpu.CompilerParams(needs_layout_passes=False),
    scratch_types=(
        pltpu.VMEM((NBUF, 128), jnp.int32),
        pltpu.VMEM((NBUF, 128), jnp.int32),
        pltpu.VMEM((NBUF * 128, FDIM), jnp.float32),
        pltpu.VMEM((ZCH, FDIM), jnp.float32),
        pltpu.SemaphoreType.DMA,
        pltpu.SemaphoreType.DMA,
        pltpu.SemaphoreType.DMA,
        pltpu.VMEM_SHARED((N_NODES, FDIM), jnp.float32),
    ),
)()


EPW = E_EDGES // NW


def _cnt_body(dst1, zcnt, fones,
              cnt_out,
              drow, cnt_v, ones_v):
    cid = lax.axis_index("c")
    sid = lax.axis_index("s")
    wid = sid * NC + cid

    pltpu.sync_copy(zcnt, cnt_v)
    pltpu.sync_copy(fones, ones_v)
    eoff = pl.multiple_of(wid * EPW, 8)
    pltpu.sync_copy(dst1.at[pl.ds(eoff, EPW)], drow)
    ones = ones_v[...]
    for g in range(EPW // 16):
        ids = drow[pl.ds(g * 16, 16)]
        plsc.addupdate_scatter(cnt_v, [ids], ones)
    ooff = pl.multiple_of(wid * N_NODES, 8)
    pltpu.sync_copy(cnt_v, cnt_out.at[pl.ds(ooff, N_NODES)])


_cnt_kernel = functools.partial(
    pl.kernel, _cnt_body,
    out_type=jax.ShapeDtypeStruct((NW * N_NODES,), jnp.float32),
    mesh=_MESH,
    compiler_params=pltpu.CompilerParams(needs_layout_passes=False),
    scratch_types=(
        pltpu.VMEM((EPW,), jnp.int32),
        pltpu.VMEM((N_NODES,), jnp.float32),
        pltpu.VMEM((16,), jnp.float32),
    ),
)()


def _escore_body(suv_h, src1, dst1, idxtab_h, zrow_h,
                 out_h,
                 suv_v, srow, drow, outv, idx_v, zrow_v, semw, semo):
    cid = lax.axis_index("c")
    sid = lax.axis_index("s")
    wid = sid * NC + cid

    pltpu.sync_copy(suv_h, suv_v)
    pltpu.sync_copy(idxtab_h, idx_v)
    pltpu.sync_copy(zrow_h, zrow_v)
    zv = zrow_v[...]
    for q in range(2):
        for r in range(3, 8):
            for g in range(8):
                outv[q * 8 + r, pl.ds(g * 16, 16)] = zv

    def chunk2(kk, carry):
        base = wid + kk * (2 * NW)
        for q in range(2):
            t = base + q * NW

            @pl.when(t < EROWS)
            def _(t=t, q=q):
                eoff = pl.multiple_of(t * 128, 128)
                pltpu.async_copy(src1.at[pl.ds(eoff, 128)], srow.at[q], semw)
                pltpu.async_copy(dst1.at[pl.ds(eoff, 128)], drow.at[q], semw)
        for q in range(2):
            t = base + q * NW

            @pl.when(t < EROWS)
            def _(t=t, q=q):
                eoff = pl.multiple_of(t * 128, 128)
                pltpu.make_async_copy(
                    src1.at[pl.ds(eoff, 128)], srow.at[q], semw).wait()
                pltpu.make_async_copy(
                    dst1.at[pl.ds(eoff, 128)], drow.at[q], semw).wait()
                eight = idx_v[pl.ds(6 * 16, 16)]
                for g in range(8):
                    sids = srow[q, pl.ds(g * 16, 16)]
                    dids = drow[q, pl.ds(g * 16, 16)]
                    s8 = sids * eight
                    d8 = dids * eight
                    for c in range(3):
                        cu = idx_v[pl.ds(c * 16, 16)]
                        cv = idx_v[pl.ds((3 + c) * 16, 16)]
                        a = plsc.load_gather(suv_v, [s8 + cu])
                        b = plsc.load_gather(suv_v, [d8 + cv])
                        outv[q * 8 + c, pl.ds(g * 16, 16)] = a + b
                toff = pl.multiple_of(t * 128, 128)
                pltpu.async_copy(outv.at[pl.ds(q * 8, 8)],
                                 out_h.at[:, pl.ds(toff, 128)], semo)
        for q in range(2):
            t = base + q * NW

            @pl.when(t < EROWS)
            def _(t=t, q=q):
                toff = pl.multiple_of(t * 128, 128)
                pltpu.make_async_copy(
                    outv.at[pl.ds(q * 8, 8)],
                    out_h.at[:, pl.ds(toff, 128)], semo).wait()
        return carry

    lax.fori_loop(0, -(-CHUNKS // 2), chunk2, 0)


_escore = functools.partial(
    pl.kernel, _escore_body,
    out_type=jax.ShapeDtypeStruct((8, E_EDGES), jnp.float32),
    mesh=_MESH,
    compiler_params=pltpu.CompilerParams(needs_layout_passes=False),
    scratch_types=(
        pltpu.VMEM((N_NODES * 8,), jnp.float32),
        pltpu.VMEM((2, 128), jnp.int32),
        pltpu.VMEM((2, 128), jnp.int32),
        pltpu.VMEM((16, 128), jnp.float32),
        pltpu.VMEM((7 * 16,), jnp.int32),
        pltpu.VMEM((16,), jnp.float32),
        pltpu.SemaphoreType.DMA,
        pltpu.SemaphoreType.DMA,
    ),
)()


BLK = 400


def _tc1_body(x_ref, acc_ref, cnt_ref, ws_ref, wn_ref, b_ref, o_ref):
    agg = acc_ref[0] + acc_ref[1]
    cnt = jnp.sum(cnt_ref[...], axis=1)[:, None]
    mean = agg / jnp.maximum(cnt, 1.0)
    h = (jnp.dot(x_ref[...], ws_ref[...], preferred_element_type=jnp.float32)
         + jnp.dot(mean, wn_ref[...], preferred_element_type=jnp.float32)
         + b_ref[...])
    o_ref[...] = jnp.maximum(h, 0.0)


def _tc2_body(h_ref, acc_ref, cnt_ref, ws_ref, wn_ref, b_ref, wc_ref, b8_ref,
              o_ref):
    agg = acc_ref[0] + acc_ref[1]
    cnt = jnp.sum(cnt_ref[...], axis=1)[:, None]
    mean = agg / jnp.maximum(cnt, 1.0)
    h2 = (jnp.dot(h_ref[...], ws_ref[...], preferred_element_type=jnp.float32)
          + jnp.dot(mean, wn_ref[...], preferred_element_type=jnp.float32)
          + b_ref[...])
    o_ref[...] = (jnp.dot(h2, wc_ref[...], preferred_element_type=jnp.float32)
                  + b8_ref[...])


def _tc1(x, acc, cnt, ws, wn, b):
    return pl.pallas_call(
        _tc1_body,
        grid=(N_NODES // BLK,),
        in_specs=[
            pl.BlockSpec((BLK, FDIM), lambda i: (i, 0)),
            pl.BlockSpec((NC, BLK, FDIM), lambda i: (0, i, 0)),
            pl.BlockSpec((BLK, NW), lambda i: (i, 0)),
            pl.BlockSpec((FDIM, FDIM), lambda i: (0, 0)),
            pl.BlockSpec((FDIM, FDIM), lambda i: (0, 0)),
            pl.BlockSpec((1, FDIM), lambda i: (0, 0)),
        ],
        out_specs=pl.BlockSpec((BLK, FDIM), lambda i: (i, 0)),
        out_shape=jax.ShapeDtypeStruct((N_NODES, FDIM), jnp.float32),
    )(x, acc, cnt, ws, wn, b)


def _tc2(h, acc, cnt, ws, wn, b, wc, b8):
    return pl.pallas_call(
        _tc2_body,
        grid=(N_NODES // BLK,),
        in_specs=[
            pl.BlockSpec((BLK, FDIM), lambda i: (i, 0)),
            pl.BlockSpec((NC, BLK, FDIM), lambda i: (0, i, 0)),
            pl.BlockSpec((BLK, NW), lambda i: (i, 0)),
            pl.BlockSpec((FDIM, FDIM), lambda i: (0, 0)),
            pl.BlockSpec((FDIM, FDIM), lambda i: (0, 0)),
            pl.BlockSpec((1, FDIM), lambda i: (0, 0)),
            pl.BlockSpec((FDIM, 8), lambda i: (0, 0)),
            pl.BlockSpec((1, 8), lambda i: (0, 0)),
        ],
        out_specs=pl.BlockSpec((BLK, 8), lambda i: (i, 0)),
        out_shape=jax.ShapeDtypeStruct((N_NODES, 8), jnp.float32),
    )(h, acc, cnt, ws, wn, b, wc, b8)


FBLK = 6400


def _fin_body(c_ref, eye_ref, o_ref):
    o_ref[...] = jax.lax.dot_general(
        c_ref[...], eye_ref[...], (((0,), (0,)), ((), ())),
        preferred_element_type=jnp.float32)


def _fin(cols, eye83):
    return pl.pallas_call(
        _fin_body,
        grid=(E_EDGES // FBLK,),
        in_specs=[
            pl.BlockSpec((8, FBLK), lambda i: (0, i)),
            pl.BlockSpec((8, 3), lambda i: (0, 0)),
        ],
        out_specs=pl.BlockSpec((FBLK, 3), lambda i: (i, 0)),
        out_shape=jax.ShapeDtypeStruct((E_EDGES, 3), jnp.float32),
    )(cols, eye83)


def kernel(x, edge_index, e, W1_self, W1_neigh, b1, W2_self, W2_neigh, b2,
           Wp, bp):
    del e
    src1 = edge_index[0]
    dst1 = edge_index[1]
    z128 = jnp.zeros((ZCH, FDIM), jnp.float32)
    zcnt = jnp.zeros((N_NODES,), jnp.float32)
    fones = jnp.ones((16,), jnp.float32)
    idxtab = jnp.asarray(
        [c for c in range(3) for _ in range(16)]
        + [4 + c for c in range(3) for _ in range(16)]
        + [8] * 16, jnp.int32)

    cnt = _cnt_kernel(dst1, zcnt, fones).reshape(NW, N_NODES).T
    acc1 = _segsum(x, src1, dst1, z128).reshape(NC, N_NODES, FDIM)
    h1 = _tc1(x, acc1, cnt, W1_self, W1_neigh, b1.reshape(1, FDIM))
    acc2 = _segsum(h1, src1, dst1, z128).reshape(NC, N_NODES, FDIM)

    wu = jnp.pad(Wp[:FDIM], ((0, 0), (0, 1)))
    wv = jnp.pad(Wp[FDIM:], ((0, 0), (0, 1)))
    wc = jnp.concatenate([wu, wv], axis=1)
    b8 = jnp.concatenate([jnp.zeros((4,), jnp.float32), bp,
                          jnp.zeros((1,), jnp.float32)]).reshape(1, 8)
    suv = _tc2(h1, acc2, cnt, W2_self, W2_neigh, b2.reshape(1, FDIM), wc, b8)

    zrow = jnp.zeros((16,), jnp.float32)
    cols = _escore(suv.reshape(-1), src1, dst1, idxtab, zrow)
    return cols[:3, :].T


# speedup vs baseline: 10.7400x; 1.0001x over previous
import functools

import jax
import jax.numpy as jnp
from jax import lax
from jax.experimental import pallas as pl
from jax.experimental.pallas import tpu as pltpu
from jax.experimental.pallas import tpu_sc as plsc

N_NODES = 10000
E_EDGES = 320000
FDIM = 128
EROWS = E_EDGES // 128
NC, NS = 2, 16
NW = NC * NS
CHUNKS = -(-EROWS // NW)
RPT = 640
ZCH = 80

_MESH = plsc.VectorSubcoreMesh(
    core_axis_name="c", subcore_axis_name="s", num_cores=NC, num_subcores=NS)


def _node_slices(sid):
    r0 = sid * RPT
    nrows = jnp.where(sid == NS - 1, 400, RPT)
    return r0, nrows


NBUF = 2


def _segsum_body(table, src1, dst1, z128,
                 acc_out,
                 srow, drow, rows, zbuf, semi, semg, sems,
                 acc_sh):
    cid = lax.axis_index("c")
    sid = lax.axis_index("s")
    wid = sid * NC + cid
    r0, nrows = _node_slices(sid)

    pltpu.sync_copy(z128, zbuf)
    for i in range(RPT // ZCH):
        @pl.when(i * ZCH < nrows)
        def _():
            off = pl.multiple_of(r0 + i * ZCH, 8)
            pltpu.sync_copy(zbuf, acc_sh.at[pl.ds(off, ZCH)])
    plsc.subcore_barrier()

    def chunk4(kk, carry):
        base = wid + kk * (NBUF * NW)
        for q in range(NBUF):
            t = base + q * NW

            @pl.when(t < EROWS)
            def _(t=t, q=q):
                eoff = pl.multiple_of(t * 128, 128)
                pltpu.async_copy(src1.at[pl.ds(eoff, 128)], srow.at[q], semi)
                pltpu.async_copy(dst1.at[pl.ds(eoff, 128)], drow.at[q], semi)
        for q in range(NBUF):
            t = base + q * NW

            @pl.when(t < EROWS)
            def _(t=t, q=q):
                eoff = pl.multiple_of(t * 128, 128)
                pltpu.make_async_copy(
                    src1.at[pl.ds(eoff, 128)], srow.at[q], semi).wait()
                pltpu.make_async_copy(
                    dst1.at[pl.ds(eoff, 128)], drow.at[q], semi).wait()
                pltpu.async_copy(table.at[srow.at[q]],
                                 rows.at[pl.ds(q * 128, 128)], semg)
        for q in range(NBUF):
            t = base + q * NW

            @pl.when(t < EROWS)
            def _(t=t, q=q):
                pltpu.make_async_copy(
                    table.at[srow.at[q]],
                    rows.at[pl.ds(q * 128, 128)], semg).wait()
                pltpu.async_copy(rows.at[pl.ds(q * 128, 128)],
                                 acc_sh.at[drow.at[q]], sems, add=True)
        for q in range(NBUF):
            t = base + q * NW

            @pl.when(t < EROWS)
            def _(t=t, q=q):
                pltpu.make_async_copy(
                    rows.at[pl.ds(q * 128, 128)],
                    acc_sh.at[drow.at[q]], sems).wait()
        return carry

    lax.fori_loop(0, -(-CHUNKS // NBUF), chunk4, 0)
    plsc.subcore_barrier()

    for i in range(RPT // ZCH):
        @pl.when(i * ZCH < nrows)
        def _():
            off = pl.multiple_of(r0 + i * ZCH, 8)
            hoff = pl.multiple_of(cid * N_NODES + r0 + i * ZCH, 8)
            pltpu.sync_copy(acc_sh.at[pl.ds(off, ZCH)], zbuf)
            pltpu.sync_copy(zbuf, acc_out.at[pl.ds(hoff, ZCH)])


_segsum = functools.partial(
    pl.kernel, _segsum_body,
    out_type=jax.ShapeDtypeStruct((NC * N_NODES, FDIM), jnp.float32),
    mesh=_MESH,
    compiler_params=pltpu.CompilerParams(needs_layout_passes=False),
    scratch_types=(
        pltpu.VMEM((NBUF, 128), jnp.int32),
        pltpu.VMEM((NBUF, 128), jnp.int32),
        pltpu.VMEM((NBUF * 128, FDIM), jnp.float32),
        pltpu.VMEM((ZCH, FDIM), jnp.float32),
        pltpu.SemaphoreType.DMA,
        pltpu.SemaphoreType.DMA,
        pltpu.SemaphoreType.DMA,
        pltpu.VMEM_SHARED((N_NODES, FDIM), jnp.float32),
    ),
)()


EPW = E_EDGES // NW


def _cnt_body(dst1, zcnt, fones,
              cnt_out,
              drow, cnt_v, ones_v):
    cid = lax.axis_index("c")
    sid = lax.axis_index("s")
    wid = sid * NC + cid

    pltpu.sync_copy(zcnt, cnt_v)
    pltpu.sync_copy(fones, ones_v)
    eoff = pl.multiple_of(wid * EPW, 8)
    pltpu.sync_copy(dst1.at[pl.ds(eoff, EPW)], drow)
    ones = ones_v[...]
    for g in range(EPW // 16):
        ids = drow[pl.ds(g * 16, 16)]
        plsc.addupdate_scatter(cnt_v, [ids], ones)
    ooff = pl.multiple_of(wid * N_NODES, 8)
    pltpu.sync_copy(cnt_v, cnt_out.at[pl.ds(ooff, N_NODES)])


_cnt_kernel = functools.partial(
    pl.kernel, _cnt_body,
    out_type=jax.ShapeDtypeStruct((NW * N_NODES,), jnp.float32),
    mesh=_MESH,
    compiler_params=pltpu.CompilerParams(needs_layout_passes=False),
    scratch_types=(
        pltpu.VMEM((EPW,), jnp.int32),
        pltpu.VMEM((N_NODES,), jnp.float32),
        pltpu.VMEM((16,), jnp.float32),
    ),
)()


def _escore_body(suv_h, src1, dst1, idxtab_h, zrow_h,
                 out_h,
                 suv_v, srow, drow, outv, idx_v, zrow_v, semw, semo):
    cid = lax.axis_index("c")
    sid = lax.axis_index("s")
    wid = sid * NC + cid

    pltpu.sync_copy(suv_h, suv_v)
    pltpu.sync_copy(idxtab_h, idx_v)
    pltpu.sync_copy(zrow_h, zrow_v)
    zv = zrow_v[...]
    for q in range(2):
        for r in range(3, 8):
            for g in range(8):
                outv[q * 8 + r, pl.ds(g * 16, 16)] = zv

    def chunk2(kk, carry):
        base = wid + kk * (2 * NW)
        for q in range(2):
            t = base + q * NW

            @pl.when(t < EROWS)
            def _(t=t, q=q):
                eoff = pl.multiple_of(t * 128, 128)
                pltpu.async_copy(src1.at[pl.ds(eoff, 128)], srow.at[q], semw)
                pltpu.async_copy(dst1.at[pl.ds(eoff, 128)], drow.at[q], semw)
        for q in range(2):
            t = base + q * NW

            @pl.when(t < EROWS)
            def _(t=t, q=q):
                eoff = pl.multiple_of(t * 128, 128)
                pltpu.make_async_copy(
                    src1.at[pl.ds(eoff, 128)], srow.at[q], semw).wait()
                pltpu.make_async_copy(
                    dst1.at[pl.ds(eoff, 128)], drow.at[q], semw).wait()
                eight = idx_v[pl.ds(6 * 16, 16)]
                for g in range(8):
                    sids = srow[q, pl.ds(g * 16, 16)]
                    dids = drow[q, pl.ds(g * 16, 16)]
                    s8 = sids * eight
                    d8 = dids * eight
                    for c in range(3):
                        cu = idx_v[pl.ds(c * 16, 16)]
                        cv = idx_v[pl.ds((3 + c) * 16, 16)]
                        a = plsc.load_gather(suv_v, [s8 + cu])
                        b = plsc.load_gather(suv_v, [d8 + cv])
                        outv[q * 8 + c, pl.ds(g * 16, 16)] = a + b
                toff = pl.multiple_of(t * 128, 128)
                pltpu.async_copy(outv.at[pl.ds(q * 8, 8)],
                                 out_h.at[:, pl.ds(toff, 128)], semo)
        for q in range(2):
            t = base + q * NW

            @pl.when(t < EROWS)
            def _(t=t, q=q):
                toff = pl.multiple_of(t * 128, 128)
                pltpu.make_async_copy(
                    outv.at[pl.ds(q * 8, 8)],
                    out_h.at[:, pl.ds(toff, 128)], semo).wait()
        return carry

    lax.fori_loop(0, -(-CHUNKS // 2), chunk2, 0)


_escore = functools.partial(
    pl.kernel, _escore_body,
    out_type=jax.ShapeDtypeStruct((8, E_EDGES), jnp.float32),
    mesh=_MESH,
    compiler_params=pltpu.CompilerParams(needs_layout_passes=False),
    scratch_types=(
        pltpu.VMEM((N_NODES * 8,), jnp.float32),
        pltpu.VMEM((2, 128), jnp.int32),
        pltpu.VMEM((2, 128), jnp.int32),
        pltpu.VMEM((16, 128), jnp.float32),
        pltpu.VMEM((7 * 16,), jnp.int32),
        pltpu.VMEM((16,), jnp.float32),
        pltpu.SemaphoreType.DMA,
        pltpu.SemaphoreType.DMA,
    ),
)()


BLK = 400


def _tc1_body(x_ref, acc_ref, cnt_ref, ws_ref, wn_ref, b_ref, o_ref):
    agg = acc_ref[0] + acc_ref[1]
    cnt = jnp.sum(cnt_ref[...], axis=1)[:, None]
    mean = agg / jnp.maximum(cnt, 1.0)
    h = (jnp.dot(x_ref[...], ws_ref[...], preferred_element_type=jnp.float32)
         + jnp.dot(mean, wn_ref[...], preferred_element_type=jnp.float32)
         + b_ref[...])
    o_ref[...] = jnp.maximum(h, 0.0)


def _tc2_body(h_ref, acc_ref, cnt_ref, ws_ref, wn_ref, b_ref, wc_ref, b8_ref,
              o_ref):
    agg = acc_ref[0] + acc_ref[1]
    cnt = jnp.sum(cnt_ref[...], axis=1)[:, None]
    mean = agg / jnp.maximum(cnt, 1.0)
    h2 = (jnp.dot(h_ref[...], ws_ref[...], preferred_element_type=jnp.float32)
          + jnp.dot(mean, wn_ref[...], preferred_element_type=jnp.float32)
          + b_ref[...])
    o_ref[...] = (jnp.dot(h2, wc_ref[...], preferred_element_type=jnp.float32)
                  + b8_ref[...])


def _tc1(x, acc, cnt, ws, wn, b):
    return pl.pallas_call(
        _tc1_body,
        grid=(N_NODES // BLK,),
        in_specs=[
            pl.BlockSpec((BLK, FDIM), lambda i: (i, 0)),
            pl.BlockSpec((NC, BLK, FDIM), lambda i: (0, i, 0)),
            pl.BlockSpec((BLK, NW), lambda i: (i, 0)),
            pl.BlockSpec((FDIM, FDIM), lambda i: (0, 0)),
            pl.BlockSpec((FDIM, FDIM), lambda i: (0, 0)),
            pl.BlockSpec((1, FDIM), lambda i: (0, 0)),
        ],
        out_specs=pl.BlockSpec((BLK, FDIM), lambda i: (i, 0)),
        out_shape=jax.ShapeDtypeStruct((N_NODES, FDIM), jnp.float32),
    )(x, acc, cnt, ws, wn, b)


def _tc2(h, acc, cnt, ws, wn, b, wc, b8):
    return pl.pallas_call(
        _tc2_body,
        grid=(N_NODES // BLK,),
        in_specs=[
            pl.BlockSpec((BLK, FDIM), lambda i: (i, 0)),
            pl.BlockSpec((NC, BLK, FDIM), lambda i: (0, i, 0)),
            pl.BlockSpec((BLK, NW), lambda i: (i, 0)),
            pl.BlockSpec((FDIM, FDIM), lambda i: (0, 0)),
            pl.BlockSpec((FDIM, FDIM), lambda i: (0, 0)),
            pl.BlockSpec((1, FDIM), lambda i: (0, 0)),
            pl.BlockSpec((FDIM, 8), lambda i: (0, 0)),
            pl.BlockSpec((1, 8), lambda i: (0, 0)),
        ],
        out_specs=pl.BlockSpec((BLK, 8), lambda i: (i, 0)),
        out_shape=jax.ShapeDtypeStruct((N_NODES, 8), jnp.float32),
    )(h, acc, cnt, ws, wn, b, wc, b8)


def kernel(x, edge_index, e, W1_self, W1_neigh, b1, W2_self, W2_neigh, b2,
           Wp, bp):
    del e
    src1 = edge_index[0]
    dst1 = edge_index[1]
    z128 = jnp.zeros((ZCH, FDIM), jnp.float32)
    zcnt = jnp.zeros((N_NODES,), jnp.float32)
    fones = jnp.ones((16,), jnp.float32)
    idxtab = jnp.asarray(
        [c for c in range(3) for _ in range(16)]
        + [4 + c for c in range(3) for _ in range(16)]
        + [8] * 16, jnp.int32)

    cnt = _cnt_kernel(dst1, zcnt, fones).reshape(NW, N_NODES).T
    acc1 = _segsum(x, src1, dst1, z128).reshape(NC, N_NODES, FDIM)
    h1 = _tc1(x, acc1, cnt, W1_self, W1_neigh, b1.reshape(1, FDIM))
    acc2 = _segsum(h1, src1, dst1, z128).reshape(NC, N_NODES, FDIM)

    wu = jnp.pad(Wp[:FDIM], ((0, 0), (0, 1)))
    wv = jnp.pad(Wp[FDIM:], ((0, 0), (0, 1)))
    wc = jnp.concatenate([wu, wv], axis=1)
    b8 = jnp.concatenate([jnp.zeros((4,), jnp.float32), bp,
                          jnp.zeros((1,), jnp.float32)]).reshape(1, 8)
    suv = _tc2(h1, acc2, cnt, W2_self, W2_neigh, b2.reshape(1, FDIM), wc, b8)

    zrow = jnp.zeros((16,), jnp.float32)
    cols = _escore(suv.reshape(-1), src1, dst1, idxtab, zrow)
    return cols[:3, :].T
